# Initial kernel scaffold; baseline (speedup 1.0000x reference)
import jax
import jax.numpy as jnp
from jax.experimental import pallas as pl


def kernel(x, edge_index, W1, b1, W2, b2, W3, b3):
    raise NotImplementedError("write your pallas kernel here")


# speedup vs baseline: 10.1920x; 10.1920x over previous
import functools

import jax
import jax.numpy as jnp
from jax import lax
from jax.experimental import pallas as pl
from jax.experimental.pallas import tpu as pltpu, tpu_sc as plsc

NC = 2
NS = 16
LANES = 16

NP = 10240
ROWS_PER_TILE = NP // NS
CHUNK = 128
CHUNKS_PER_TILE = 40
EP = NC * NS * CHUNKS_PER_TILE * CHUNK

_MESH = plsc.VectorSubcoreMesh(core_axis_name="c", subcore_axis_name="s")


def _deg_body(dst_hbm, out_hbm, dst_v, deg_v, acc_v, tmp_v, shared):
    c = lax.axis_index("c")
    s = lax.axis_index("s")
    w = c * NS + s

    pltpu.sync_copy(dst_hbm.at[w], dst_v)

    zeros16 = jnp.zeros((LANES,), jnp.float32)

    def _zero(j, _):
        deg_v[pl.ds(j * LANES, LANES)] = zeros16
        return 0

    lax.fori_loop(0, NP // LANES, _zero, 0)

    ones16 = jnp.ones((LANES,), jnp.float32)

    def _edges(j, _):
        for k in range(CHUNK // LANES):
            d = dst_v[j, pl.ds(k * LANES, LANES)]
            plsc.addupdate_scatter(deg_v, [d], ones16)
        return 0

    lax.fori_loop(0, CHUNKS_PER_TILE, _edges, 0)

    pltpu.sync_copy(deg_v, shared.at[s])
    plsc.subcore_barrier()

    base = s * ROWS_PER_TILE

    def _zacc(j, _):
        acc_v[pl.ds(j * LANES, LANES)] = zeros16
        return 0

    lax.fori_loop(0, ROWS_PER_TILE // LANES, _zacc, 0)

    for t in range(NS):
        pltpu.sync_copy(shared.at[t, pl.ds(base, ROWS_PER_TILE)], tmp_v)

        def _acc(j, _):
            sl = pl.ds(j * LANES, LANES)
            acc_v[sl] = acc_v[sl] + tmp_v[sl]
            return 0

        lax.fori_loop(0, ROWS_PER_TILE // LANES, _acc, 0)

    pltpu.sync_copy(acc_v, out_hbm.at[c, pl.ds(base, ROWS_PER_TILE)])


_deg_kernel = pl.kernel(
    _deg_body,
    out_type=jax.ShapeDtypeStruct((NC, NP), jnp.float32),
    mesh=_MESH,
    scratch_types=[
        pltpu.VMEM((CHUNKS_PER_TILE, CHUNK), jnp.int32),
        pltpu.VMEM((NP,), jnp.float32),
        pltpu.VMEM((ROWS_PER_TILE,), jnp.float32),
        pltpu.VMEM((ROWS_PER_TILE,), jnp.float32),
        pltpu.VMEM_SHARED((NS, NP), jnp.float32),
    ],
    compiler_params=pltpu.CompilerParams(
        needs_layout_passes=False, use_tc_tiling_on_sc=False
    ),
)


def _edge_body(g_hbm, src_hbm, dst_hbm, out_hbm, src_v, dst_v, rows_v, zbuf_v,
               shared, *, h):
    c = lax.axis_index("c")
    s = lax.axis_index("s")
    w = c * NS + s

    pltpu.sync_copy(src_hbm.at[w], src_v)
    pltpu.sync_copy(dst_hbm.at[w], dst_v)

    zeros16 = jnp.zeros((LANES,), jnp.float32)

    def _zero(j, _):
        for k in range(h // LANES):
            zbuf_v[j, pl.ds(k * LANES, LANES)] = zeros16
        return 0

    lax.fori_loop(0, CHUNK, _zero, 0)

    base = s * ROWS_PER_TILE
    for i in range(ROWS_PER_TILE // CHUNK):
        pltpu.sync_copy(zbuf_v, shared.at[pl.ds(base + i * CHUNK, CHUNK)])
    plsc.subcore_barrier()

    def _chunk(j, _):
        pltpu.sync_copy(g_hbm.at[src_v.at[j]], rows_v)
        pltpu.sync_copy(rows_v, shared.at[dst_v.at[j]], add=True)
        return 0

    lax.fori_loop(0, CHUNKS_PER_TILE, _chunk, 0)

    plsc.subcore_barrier()
    for i in range(ROWS_PER_TILE // CHUNK):
        pltpu.sync_copy(
            shared.at[pl.ds(base + i * CHUNK, CHUNK)],
            out_hbm.at[c, pl.ds(base + i * CHUNK, CHUNK)],
        )


@functools.cache
def _edge_kernel(h):
    return pl.kernel(
        functools.partial(_edge_body, h=h),
        out_type=jax.ShapeDtypeStruct((NC, NP, h), jnp.float32),
        mesh=_MESH,
        scratch_types=[
            pltpu.VMEM((CHUNKS_PER_TILE, CHUNK), jnp.int32),
            pltpu.VMEM((CHUNKS_PER_TILE, CHUNK), jnp.int32),
            pltpu.VMEM((CHUNK, h), jnp.float32),
            pltpu.VMEM((CHUNK, h), jnp.float32),
            pltpu.VMEM_SHARED((NP, h), jnp.float32),
        ],
        compiler_params=pltpu.CompilerParams(use_tc_tiling_on_sc=False),
    )


_BLK = 2048
_GRID = NP // _BLK


def _k1_body(deg_ref, x_ref, w_ref, h_ref, g_ref, dinv_ref, ood_ref):
    deg = jnp.sum(deg_ref[...], axis=0) + 1.0
    dinv = lax.rsqrt(deg)
    ood = 1.0 / deg
    hmat = jnp.dot(x_ref[...], w_ref[...], preferred_element_type=jnp.float32)
    h_ref[...] = hmat
    g_ref[...] = hmat * dinv
    dinv_ref[...] = dinv
    ood_ref[...] = ood


def _tc_first(deg2, xp, W1):
    h1w = W1.shape[1]
    return pl.pallas_call(
        _k1_body,
        grid=(_GRID,),
        in_specs=[
            pl.BlockSpec((NC, _BLK, 1), lambda i: (0, i, 0)),
            pl.BlockSpec((_BLK, xp.shape[1]), lambda i: (i, 0)),
            pl.BlockSpec(W1.shape, lambda i: (0, 0)),
        ],
        out_specs=[
            pl.BlockSpec((_BLK, h1w), lambda i: (i, 0)),
            pl.BlockSpec((_BLK, h1w), lambda i: (i, 0)),
            pl.BlockSpec((_BLK, 1), lambda i: (i, 0)),
            pl.BlockSpec((_BLK, 1), lambda i: (i, 0)),
        ],
        out_shape=[
            jax.ShapeDtypeStruct((NP, h1w), jnp.float32),
            jax.ShapeDtypeStruct((NP, h1w), jnp.float32),
            jax.ShapeDtypeStruct((NP, 1), jnp.float32),
            jax.ShapeDtypeStruct((NP, 1), jnp.float32),
        ],
    )(deg2, xp, W1)


def _k2_body(parts_ref, hcur_ref, dinv_ref, ood_ref, b_ref, w_ref,
             hn_ref, gn_ref):
    dinv = dinv_ref[...]
    z = (parts_ref[0] + parts_ref[1]) * dinv
    z = z + hcur_ref[...] * ood_ref[...] + b_ref[...]
    z = jnp.maximum(z, 0.0)
    hn = jnp.dot(z, w_ref[...], preferred_element_type=jnp.float32)
    hn_ref[...] = hn
    gn_ref[...] = hn * dinv


def _tc_mid(parts, hcur, dinv, ood, b, Wn):
    hw = hcur.shape[1]
    nw = Wn.shape[1]
    return pl.pallas_call(
        _k2_body,
        grid=(_GRID,),
        in_specs=[
            pl.BlockSpec((NC, _BLK, hw), lambda i: (0, i, 0)),
            pl.BlockSpec((_BLK, hw), lambda i: (i, 0)),
            pl.BlockSpec((_BLK, 1), lambda i: (i, 0)),
            pl.BlockSpec((_BLK, 1), lambda i: (i, 0)),
            pl.BlockSpec((1, hw), lambda i: (0, 0)),
            pl.BlockSpec((hw, nw), lambda i: (0, 0)),
        ],
        out_specs=[
            pl.BlockSpec((_BLK, nw), lambda i: (i, 0)),
            pl.BlockSpec((_BLK, nw), lambda i: (i, 0)),
        ],
        out_shape=[
            jax.ShapeDtypeStruct((NP, nw), jnp.float32),
            jax.ShapeDtypeStruct((NP, nw), jnp.float32),
        ],
    )(parts, hcur, dinv, ood, b, Wn)


def _k3_body(parts_ref, hcur_ref, dinv_ref, ood_ref, b_ref, out_ref, *, valid):
    logits = (parts_ref[0] + parts_ref[1]) * dinv_ref[...]
    logits = logits + hcur_ref[...] * ood_ref[...] + b_ref[...]
    cols = lax.broadcasted_iota(jnp.int32, logits.shape, 1)
    logits = jnp.where(cols < valid, logits, -1e30)
    m = jnp.max(logits, axis=-1, keepdims=True)
    e = jnp.exp(logits - m)
    out_ref[...] = e / jnp.sum(e, axis=-1, keepdims=True)


def _tc_last(parts, hcur, dinv, ood, b, valid):
    hw = hcur.shape[1]
    return pl.pallas_call(
        functools.partial(_k3_body, valid=valid),
        grid=(_GRID,),
        in_specs=[
            pl.BlockSpec((NC, _BLK, hw), lambda i: (0, i, 0)),
            pl.BlockSpec((_BLK, hw), lambda i: (i, 0)),
            pl.BlockSpec((_BLK, 1), lambda i: (i, 0)),
            pl.BlockSpec((_BLK, 1), lambda i: (i, 0)),
            pl.BlockSpec((1, hw), lambda i: (0, 0)),
        ],
        out_specs=pl.BlockSpec((_BLK, hw), lambda i: (i, 0)),
        out_shape=jax.ShapeDtypeStruct((NP, hw), jnp.float32),
    )(parts, hcur, dinv, ood, b)


def kernel(x, edge_index, W1, b1, W2, b2, W3, b3):
    n, _ = x.shape
    e = edge_index.shape[1]

    xp = jnp.pad(x, ((0, NP - n), (0, 0)))
    pad_e = EP - e
    src3 = jnp.concatenate(
        [edge_index[0], jnp.full((pad_e,), n, jnp.int32)]
    ).reshape(NC * NS, CHUNKS_PER_TILE, CHUNK)
    dst3 = jnp.concatenate(
        [edge_index[1], jnp.full((pad_e,), n, jnp.int32)]
    ).reshape(NC * NS, CHUNKS_PER_TILE, CHUNK)

    nclass = W3.shape[1]
    W3p = jnp.pad(W3, ((0, 0), (0, 48 - nclass)))
    b3p = jnp.pad(b3, (0, 48 - nclass))

    deg_parts = _deg_kernel(dst3)
    deg2 = deg_parts.reshape(NC, NP, 1)

    h1, g1, dinv, ood = _tc_first(deg2, xp, W1)
    parts1 = _edge_kernel(W1.shape[1])(g1, src3, dst3)
    h2, g2 = _tc_mid(parts1, h1, dinv, ood, b1.reshape(1, -1), W2)
    parts2 = _edge_kernel(W2.shape[1])(g2, src3, dst3)
    h3, g3 = _tc_mid(parts2, h2, dinv, ood, b2.reshape(1, -1), W3p)
    parts3 = _edge_kernel(48)(g3, src3, dst3)
    out = _tc_last(parts3, h3, dinv, ood, b3p.reshape(1, -1), nclass)
    return out[:n, :nclass]


# speedup vs baseline: 11.9916x; 1.1766x over previous
import functools

import jax
import jax.numpy as jnp
from jax import lax
from jax.experimental import pallas as pl
from jax.experimental.pallas import tpu as pltpu, tpu_sc as plsc

NC = 2
NS = 16
LANES = 16

NP = 10240
ROWS_PER_TILE = NP // NS
CHUNK = 128
CHUNKS_PER_TILE = 40
EP = NC * NS * CHUNKS_PER_TILE * CHUNK

_MESH = plsc.VectorSubcoreMesh(core_axis_name="c", subcore_axis_name="s")


def _deg_body(dst_hbm, out_hbm, dst_v, deg_v, acc_v, tmp_v, shared):
    c = lax.axis_index("c")
    s = lax.axis_index("s")
    w = c * NS + s

    pltpu.sync_copy(dst_hbm.at[w], dst_v)

    zeros16 = jnp.zeros((LANES,), jnp.float32)

    def _zero(j, _):
        deg_v[pl.ds(j * LANES, LANES)] = zeros16
        return 0

    lax.fori_loop(0, NP // LANES, _zero, 0)

    ones16 = jnp.ones((LANES,), jnp.float32)

    def _edges(j, _):
        for k in range(CHUNK // LANES):
            d = dst_v[j, pl.ds(k * LANES, LANES)]
            plsc.addupdate_scatter(deg_v, [d], ones16)
        return 0

    lax.fori_loop(0, CHUNKS_PER_TILE, _edges, 0)

    pltpu.sync_copy(deg_v, shared.at[s])
    plsc.subcore_barrier()

    base = s * ROWS_PER_TILE

    def _zacc(j, _):
        acc_v[pl.ds(j * LANES, LANES)] = zeros16
        return 0

    lax.fori_loop(0, ROWS_PER_TILE // LANES, _zacc, 0)

    for t in range(NS):
        pltpu.sync_copy(shared.at[t, pl.ds(base, ROWS_PER_TILE)], tmp_v)

        def _acc(j, _):
            sl = pl.ds(j * LANES, LANES)
            acc_v[sl] = acc_v[sl] + tmp_v[sl]
            return 0

        lax.fori_loop(0, ROWS_PER_TILE // LANES, _acc, 0)

    pltpu.sync_copy(acc_v, out_hbm.at[c, pl.ds(base, ROWS_PER_TILE)])


_deg_kernel = pl.kernel(
    _deg_body,
    out_type=jax.ShapeDtypeStruct((NC, NP), jnp.float32),
    mesh=_MESH,
    scratch_types=[
        pltpu.VMEM((CHUNKS_PER_TILE, CHUNK), jnp.int32),
        pltpu.VMEM((NP,), jnp.float32),
        pltpu.VMEM((ROWS_PER_TILE,), jnp.float32),
        pltpu.VMEM((ROWS_PER_TILE,), jnp.float32),
        pltpu.VMEM_SHARED((NS, NP), jnp.float32),
    ],
    compiler_params=pltpu.CompilerParams(
        needs_layout_passes=False, use_tc_tiling_on_sc=False
    ),
)


_NBUF = 4


def _edge_body(g_hbm, src_hbm, dst_hbm, out_hbm, src_v, dst_v, rows_v, zbuf_v,
               shared, *sems, h):
    gsems = sems[:_NBUF]
    ssems = sems[_NBUF:]
    c = lax.axis_index("c")
    s = lax.axis_index("s")
    w = c * NS + s

    pltpu.sync_copy(src_hbm.at[w], src_v)
    pltpu.sync_copy(dst_hbm.at[w], dst_v)

    zeros16 = jnp.zeros((LANES,), jnp.float32)

    def _zero(j, _):
        for k in range(h // LANES):
            zbuf_v[j, pl.ds(k * LANES, LANES)] = zeros16
        return 0

    lax.fori_loop(0, CHUNK, _zero, 0)

    base = s * ROWS_PER_TILE
    for i in range(ROWS_PER_TILE // CHUNK):
        pltpu.sync_copy(zbuf_v, shared.at[pl.ds(base + i * CHUNK, CHUNK)])
    plsc.subcore_barrier()

    gd = [None] * _NBUF
    sd = [None] * _NBUF
    for b in range(_NBUF):
        gd[b] = pltpu.async_copy(g_hbm.at[src_v.at[b]], rows_v.at[b], gsems[b])
    for j in range(CHUNKS_PER_TILE):
        b = j % _NBUF
        gd[b].wait()
        sd[b] = pltpu.async_copy(
            rows_v.at[b], shared.at[dst_v.at[j]], ssems[b], add=True
        )
        if j + _NBUF < CHUNKS_PER_TILE:
            sd[b].wait()
            gd[b] = pltpu.async_copy(
                g_hbm.at[src_v.at[j + _NBUF]], rows_v.at[b], gsems[b]
            )
    for j in range(CHUNKS_PER_TILE - _NBUF, CHUNKS_PER_TILE):
        sd[j % _NBUF].wait()

    plsc.subcore_barrier()
    for i in range(ROWS_PER_TILE // CHUNK):
        pltpu.sync_copy(
            shared.at[pl.ds(base + i * CHUNK, CHUNK)],
            out_hbm.at[c, pl.ds(base + i * CHUNK, CHUNK)],
        )


@functools.cache
def _edge_kernel(h):
    return pl.kernel(
        functools.partial(_edge_body, h=h),
        out_type=jax.ShapeDtypeStruct((NC, NP, h), jnp.float32),
        mesh=_MESH,
        scratch_types=[
            pltpu.VMEM((CHUNKS_PER_TILE, CHUNK), jnp.int32),
            pltpu.VMEM((CHUNKS_PER_TILE, CHUNK), jnp.int32),
            pltpu.VMEM((_NBUF, CHUNK, h), jnp.float32),
            pltpu.VMEM((CHUNK, h), jnp.float32),
            pltpu.VMEM_SHARED((NP, h), jnp.float32),
        ]
        + [pltpu.SemaphoreType.DMA] * (2 * _NBUF),
        compiler_params=pltpu.CompilerParams(use_tc_tiling_on_sc=False),
    )


_BLK = 2048
_GRID = NP // _BLK


def _k1_body(deg_ref, x_ref, w_ref, h_ref, g_ref, dinv_ref, ood_ref):
    deg = jnp.sum(deg_ref[...], axis=0) + 1.0
    dinv = lax.rsqrt(deg)
    ood = 1.0 / deg
    hmat = jnp.dot(x_ref[...], w_ref[...], preferred_element_type=jnp.float32)
    h_ref[...] = hmat
    g_ref[...] = hmat * dinv
    dinv_ref[...] = dinv
    ood_ref[...] = ood


def _tc_first(deg2, xp, W1):
    h1w = W1.shape[1]
    return pl.pallas_call(
        _k1_body,
        grid=(_GRID,),
        in_specs=[
            pl.BlockSpec((NC, _BLK, 1), lambda i: (0, i, 0)),
            pl.BlockSpec((_BLK, xp.shape[1]), lambda i: (i, 0)),
            pl.BlockSpec(W1.shape, lambda i: (0, 0)),
        ],
        out_specs=[
            pl.BlockSpec((_BLK, h1w), lambda i: (i, 0)),
            pl.BlockSpec((_BLK, h1w), lambda i: (i, 0)),
            pl.BlockSpec((_BLK, 1), lambda i: (i, 0)),
            pl.BlockSpec((_BLK, 1), lambda i: (i, 0)),
        ],
        out_shape=[
            jax.ShapeDtypeStruct((NP, h1w), jnp.float32),
            jax.ShapeDtypeStruct((NP, h1w), jnp.float32),
            jax.ShapeDtypeStruct((NP, 1), jnp.float32),
            jax.ShapeDtypeStruct((NP, 1), jnp.float32),
        ],
    )(deg2, xp, W1)


def _k2_body(parts_ref, hcur_ref, dinv_ref, ood_ref, b_ref, w_ref,
             hn_ref, gn_ref):
    dinv = dinv_ref[...]
    z = (parts_ref[0] + parts_ref[1]) * dinv
    z = z + hcur_ref[...] * ood_ref[...] + b_ref[...]
    z = jnp.maximum(z, 0.0)
    hn = jnp.dot(z, w_ref[...], preferred_element_type=jnp.float32)
    hn_ref[...] = hn
    gn_ref[...] = hn * dinv


def _tc_mid(parts, hcur, dinv, ood, b, Wn):
    hw = hcur.shape[1]
    nw = Wn.shape[1]
    return pl.pallas_call(
        _k2_body,
        grid=(_GRID,),
        in_specs=[
            pl.BlockSpec((NC, _BLK, hw), lambda i: (0, i, 0)),
            pl.BlockSpec((_BLK, hw), lambda i: (i, 0)),
            pl.BlockSpec((_BLK, 1), lambda i: (i, 0)),
            pl.BlockSpec((_BLK, 1), lambda i: (i, 0)),
            pl.BlockSpec((1, hw), lambda i: (0, 0)),
            pl.BlockSpec((hw, nw), lambda i: (0, 0)),
        ],
        out_specs=[
            pl.BlockSpec((_BLK, nw), lambda i: (i, 0)),
            pl.BlockSpec((_BLK, nw), lambda i: (i, 0)),
        ],
        out_shape=[
            jax.ShapeDtypeStruct((NP, nw), jnp.float32),
            jax.ShapeDtypeStruct((NP, nw), jnp.float32),
        ],
    )(parts, hcur, dinv, ood, b, Wn)


def _k3_body(parts_ref, hcur_ref, dinv_ref, ood_ref, b_ref, out_ref, *, valid):
    logits = (parts_ref[0] + parts_ref[1]) * dinv_ref[...]
    logits = logits + hcur_ref[...] * ood_ref[...] + b_ref[...]
    cols = lax.broadcasted_iota(jnp.int32, logits.shape, 1)
    logits = jnp.where(cols < valid, logits, -1e30)
    m = jnp.max(logits, axis=-1, keepdims=True)
    e = jnp.exp(logits - m)
    out_ref[...] = e / jnp.sum(e, axis=-1, keepdims=True)


def _tc_last(parts, hcur, dinv, ood, b, valid):
    hw = hcur.shape[1]
    return pl.pallas_call(
        functools.partial(_k3_body, valid=valid),
        grid=(_GRID,),
        in_specs=[
            pl.BlockSpec((NC, _BLK, hw), lambda i: (0, i, 0)),
            pl.BlockSpec((_BLK, hw), lambda i: (i, 0)),
            pl.BlockSpec((_BLK, 1), lambda i: (i, 0)),
            pl.BlockSpec((_BLK, 1), lambda i: (i, 0)),
            pl.BlockSpec((1, hw), lambda i: (0, 0)),
        ],
        out_specs=pl.BlockSpec((_BLK, hw), lambda i: (i, 0)),
        out_shape=jax.ShapeDtypeStruct((NP, hw), jnp.float32),
    )(parts, hcur, dinv, ood, b)


def kernel(x, edge_index, W1, b1, W2, b2, W3, b3):
    n, _ = x.shape
    e = edge_index.shape[1]

    xp = jnp.pad(x, ((0, NP - n), (0, 0)))
    pad_e = EP - e
    src3 = jnp.concatenate(
        [edge_index[0], jnp.full((pad_e,), n, jnp.int32)]
    ).reshape(NC * NS, CHUNKS_PER_TILE, CHUNK)
    dst3 = jnp.concatenate(
        [edge_index[1], jnp.full((pad_e,), n, jnp.int32)]
    ).reshape(NC * NS, CHUNKS_PER_TILE, CHUNK)

    nclass = W3.shape[1]
    W3p = jnp.pad(W3, ((0, 0), (0, 48 - nclass)))
    b3p = jnp.pad(b3, (0, 48 - nclass))

    deg_parts = _deg_kernel(dst3)
    deg2 = deg_parts.reshape(NC, NP, 1)

    h1, g1, dinv, ood = _tc_first(deg2, xp, W1)
    parts1 = _edge_kernel(W1.shape[1])(g1, src3, dst3)
    h2, g2 = _tc_mid(parts1, h1, dinv, ood, b1.reshape(1, -1), W2)
    parts2 = _edge_kernel(W2.shape[1])(g2, src3, dst3)
    h3, g3 = _tc_mid(parts2, h2, dinv, ood, b2.reshape(1, -1), W3p)
    parts3 = _edge_kernel(48)(g3, src3, dst3)
    out = _tc_last(parts3, h3, dinv, ood, b3p.reshape(1, -1), nclass)
    return out[:n, :nclass]


# speedup vs baseline: 21.2822x; 1.7748x over previous
import functools

import jax
import jax.numpy as jnp
from jax import lax
from jax.experimental import pallas as pl
from jax.experimental.pallas import tpu as pltpu, tpu_sc as plsc

NC = 2
NS = 16
LANES = 16

NP = 10240
ROWS_PER_TILE = NP // NS
CHUNK = 128
CHUNKS_PER_TILE = 40
EP = NC * NS * CHUNKS_PER_TILE * CHUNK

_MESH = plsc.VectorSubcoreMesh(core_axis_name="c", subcore_axis_name="s")


def _deg_body(dst_hbm, out_hbm, dst_v, deg_v, acc_v, tmp_v, shared):
    c = lax.axis_index("c")
    s = lax.axis_index("s")
    w = c * NS + s

    pltpu.sync_copy(dst_hbm.at[w], dst_v)

    zeros16 = jnp.zeros((LANES,), jnp.float32)

    def _zero(j, _):
        deg_v[pl.ds(j * LANES, LANES)] = zeros16
        return 0

    lax.fori_loop(0, NP // LANES, _zero, 0)

    ones16 = jnp.ones((LANES,), jnp.float32)

    def _edges(j, _):
        for k in range(CHUNK // LANES):
            d = dst_v[j, pl.ds(k * LANES, LANES)]
            plsc.addupdate_scatter(deg_v, [d], ones16)
        return 0

    lax.fori_loop(0, CHUNKS_PER_TILE, _edges, 0)

    pltpu.sync_copy(deg_v, shared.at[s])
    plsc.subcore_barrier()

    base = s * ROWS_PER_TILE

    def _zacc(j, _):
        acc_v[pl.ds(j * LANES, LANES)] = zeros16
        return 0

    lax.fori_loop(0, ROWS_PER_TILE // LANES, _zacc, 0)

    for t in range(NS):
        pltpu.sync_copy(shared.at[t, pl.ds(base, ROWS_PER_TILE)], tmp_v)

        def _acc(j, _):
            sl = pl.ds(j * LANES, LANES)
            acc_v[sl] = acc_v[sl] + tmp_v[sl]
            return 0

        lax.fori_loop(0, ROWS_PER_TILE // LANES, _acc, 0)

    pltpu.sync_copy(acc_v, out_hbm.at[c, pl.ds(base, ROWS_PER_TILE)])


_deg_kernel = pl.kernel(
    _deg_body,
    out_type=jax.ShapeDtypeStruct((NC, NP), jnp.float32),
    mesh=_MESH,
    scratch_types=[
        pltpu.VMEM((CHUNKS_PER_TILE, CHUNK), jnp.int32),
        pltpu.VMEM((NP,), jnp.float32),
        pltpu.VMEM((ROWS_PER_TILE,), jnp.float32),
        pltpu.VMEM((ROWS_PER_TILE,), jnp.float32),
        pltpu.VMEM_SHARED((NS, NP), jnp.float32),
    ],
    compiler_params=pltpu.CompilerParams(
        needs_layout_passes=False, use_tc_tiling_on_sc=False
    ),
)


_NBUF = 4


def _edge_body(g_hbm, src_hbm, dst_hbm, out_hbm, src_v, dst_v, rows_v, zbuf_v,
               shared, *sems, h):
    gsems = sems[:_NBUF]
    ssems = sems[_NBUF:]
    c = lax.axis_index("c")
    s = lax.axis_index("s")
    w = c * NS + s

    pltpu.sync_copy(src_hbm.at[w], src_v)
    pltpu.sync_copy(dst_hbm.at[w], dst_v)

    zeros16 = jnp.zeros((LANES,), jnp.float32)

    def _zero(j, _):
        for k in range(h // LANES):
            zbuf_v[j, pl.ds(k * LANES, LANES)] = zeros16
        return 0

    lax.fori_loop(0, CHUNK, _zero, 0)

    base = s * ROWS_PER_TILE
    for i in range(ROWS_PER_TILE // CHUNK):
        pltpu.sync_copy(zbuf_v, shared.at[pl.ds(base + i * CHUNK, CHUNK)])
    plsc.subcore_barrier()

    gd = [None] * _NBUF
    sd = [None] * _NBUF
    for b in range(_NBUF):
        gd[b] = pltpu.async_copy(g_hbm.at[src_v.at[b]], rows_v.at[b], gsems[b])
    for j in range(CHUNKS_PER_TILE):
        b = j % _NBUF
        gd[b].wait()
        sd[b] = pltpu.async_copy(
            rows_v.at[b], shared.at[dst_v.at[j]], ssems[b], add=True
        )
        if j + _NBUF < CHUNKS_PER_TILE:
            sd[b].wait()
            gd[b] = pltpu.async_copy(
                g_hbm.at[src_v.at[j + _NBUF]], rows_v.at[b], gsems[b]
            )
    for j in range(CHUNKS_PER_TILE - _NBUF, CHUNKS_PER_TILE):
        sd[j % _NBUF].wait()

    plsc.subcore_barrier()
    for i in range(ROWS_PER_TILE // CHUNK):
        pltpu.sync_copy(
            shared.at[pl.ds(base + i * CHUNK, CHUNK)],
            out_hbm.at[c, pl.ds(base + i * CHUNK, CHUNK)],
        )


@functools.cache
def _edge_kernel(h):
    return pl.kernel(
        functools.partial(_edge_body, h=h),
        out_type=jax.ShapeDtypeStruct((NC, NP, h), jnp.float32),
        mesh=_MESH,
        scratch_types=[
            pltpu.VMEM((CHUNKS_PER_TILE, CHUNK), jnp.int32),
            pltpu.VMEM((CHUNKS_PER_TILE, CHUNK), jnp.int32),
            pltpu.VMEM((_NBUF, CHUNK, h), jnp.float32),
            pltpu.VMEM((CHUNK, h), jnp.float32),
            pltpu.VMEM_SHARED((NP, h), jnp.float32),
        ]
        + [pltpu.SemaphoreType.DMA] * (2 * _NBUF),
        compiler_params=pltpu.CompilerParams(use_tc_tiling_on_sc=False),
    )


_BLK = 2048
_GRID = NP // _BLK


def _k1_body(deg_ref, x_ref, w_ref, h_ref, g_ref, dinv_ref, ood_ref):
    deg = jnp.sum(deg_ref[...], axis=0) + 1.0
    dinv = lax.rsqrt(deg)
    ood = 1.0 / deg
    hmat = jnp.dot(x_ref[...], w_ref[...], preferred_element_type=jnp.float32)
    h_ref[...] = hmat
    g_ref[...] = hmat * dinv
    dinv_ref[...] = dinv
    ood_ref[...] = ood


def _tc_first(deg2, xp, W1):
    h1w = W1.shape[1]
    return pl.pallas_call(
        _k1_body,
        grid=(_GRID,),
        in_specs=[
            pl.BlockSpec((NC, _BLK, 1), lambda i: (0, i, 0)),
            pl.BlockSpec((_BLK, xp.shape[1]), lambda i: (i, 0)),
            pl.BlockSpec(W1.shape, lambda i: (0, 0)),
        ],
        out_specs=[
            pl.BlockSpec((_BLK, h1w), lambda i: (i, 0)),
            pl.BlockSpec((_BLK, h1w), lambda i: (i, 0)),
            pl.BlockSpec((_BLK, 1), lambda i: (i, 0)),
            pl.BlockSpec((_BLK, 1), lambda i: (i, 0)),
        ],
        out_shape=[
            jax.ShapeDtypeStruct((NP, h1w), jnp.float32),
            jax.ShapeDtypeStruct((NP, h1w), jnp.float32),
            jax.ShapeDtypeStruct((NP, 1), jnp.float32),
            jax.ShapeDtypeStruct((NP, 1), jnp.float32),
        ],
    )(deg2, xp, W1)


def _k2_body(parts_ref, hcur_ref, dinv_ref, ood_ref, b_ref, w_ref,
             hn_ref, gn_ref):
    dinv = dinv_ref[...]
    z = (parts_ref[0] + parts_ref[1]) * dinv
    z = z + hcur_ref[...] * ood_ref[...] + b_ref[...]
    z = jnp.maximum(z, 0.0)
    hn = jnp.dot(z, w_ref[...], preferred_element_type=jnp.float32)
    hn_ref[...] = hn
    gn_ref[...] = hn * dinv


def _tc_mid(parts, hcur, dinv, ood, b, Wn):
    hw = hcur.shape[1]
    nw = Wn.shape[1]
    return pl.pallas_call(
        _k2_body,
        grid=(_GRID,),
        in_specs=[
            pl.BlockSpec((NC, _BLK, hw), lambda i: (0, i, 0)),
            pl.BlockSpec((_BLK, hw), lambda i: (i, 0)),
            pl.BlockSpec((_BLK, 1), lambda i: (i, 0)),
            pl.BlockSpec((_BLK, 1), lambda i: (i, 0)),
            pl.BlockSpec((1, hw), lambda i: (0, 0)),
            pl.BlockSpec((hw, nw), lambda i: (0, 0)),
        ],
        out_specs=[
            pl.BlockSpec((_BLK, nw), lambda i: (i, 0)),
            pl.BlockSpec((_BLK, nw), lambda i: (i, 0)),
        ],
        out_shape=[
            jax.ShapeDtypeStruct((NP, nw), jnp.float32),
            jax.ShapeDtypeStruct((NP, nw), jnp.float32),
        ],
    )(parts, hcur, dinv, ood, b, Wn)


def _k3_body(parts_ref, hcur_ref, dinv_ref, ood_ref, b_ref, out_ref, *, valid):
    logits = (parts_ref[0] + parts_ref[1]) * dinv_ref[...]
    logits = logits + hcur_ref[...] * ood_ref[...] + b_ref[...]
    cols = lax.broadcasted_iota(jnp.int32, logits.shape, 1)
    logits = jnp.where(cols < valid, logits, -1e30)
    m = jnp.max(logits, axis=-1, keepdims=True)
    e = jnp.exp(logits - m)
    out_ref[...] = e / jnp.sum(e, axis=-1, keepdims=True)


def _tc_last(parts, hcur, dinv, ood, b, valid):
    hw = hcur.shape[1]
    return pl.pallas_call(
        functools.partial(_k3_body, valid=valid),
        grid=(_GRID,),
        in_specs=[
            pl.BlockSpec((NC, _BLK, hw), lambda i: (0, i, 0)),
            pl.BlockSpec((_BLK, hw), lambda i: (i, 0)),
            pl.BlockSpec((_BLK, 1), lambda i: (i, 0)),
            pl.BlockSpec((_BLK, 1), lambda i: (i, 0)),
            pl.BlockSpec((1, hw), lambda i: (0, 0)),
        ],
        out_specs=pl.BlockSpec((_BLK, hw), lambda i: (i, 0)),
        out_shape=jax.ShapeDtypeStruct((NP, hw), jnp.float32),
    )(parts, hcur, dinv, ood, b)


def kernel(x, edge_index, W1, b1, W2, b2, W3, b3):
    n, _ = x.shape
    e = edge_index.shape[1]

    xp = jnp.pad(x, ((0, NP - n), (0, 0)))
    nw = NC * NS
    per_w = EP // nw
    pad_w = per_w - e // nw
    pad_idx = (
        n
        + (jnp.arange(nw * pad_w, dtype=jnp.int32) * 7) % (NP - n)
    ).reshape(nw, pad_w)
    src3 = jnp.concatenate(
        [edge_index[0].reshape(nw, e // nw), pad_idx], axis=1
    ).reshape(nw, CHUNKS_PER_TILE, CHUNK)
    dst3 = jnp.concatenate(
        [edge_index[1].reshape(nw, e // nw), pad_idx], axis=1
    ).reshape(nw, CHUNKS_PER_TILE, CHUNK)

    nclass = W3.shape[1]
    W3p = jnp.pad(W3, ((0, 0), (0, 48 - nclass)))
    b3p = jnp.pad(b3, (0, 48 - nclass))

    deg_parts = _deg_kernel(dst3)
    deg2 = deg_parts.reshape(NC, NP, 1)

    h1, g1, dinv, ood = _tc_first(deg2, xp, W1)
    parts1 = _edge_kernel(W1.shape[1])(g1, src3, dst3)
    h2, g2 = _tc_mid(parts1, h1, dinv, ood, b1.reshape(1, -1), W2)
    parts2 = _edge_kernel(W2.shape[1])(g2, src3, dst3)
    h3, g3 = _tc_mid(parts2, h2, dinv, ood, b2.reshape(1, -1), W3p)
    parts3 = _edge_kernel(48)(g3, src3, dst3)
    out = _tc_last(parts3, h3, dinv, ood, b3p.reshape(1, -1), nclass)
    return out[:n, :nclass]


# speedup vs baseline: 21.4816x; 1.0094x over previous
import functools

import jax
import jax.numpy as jnp
from jax import lax
from jax.experimental import pallas as pl
from jax.experimental.pallas import tpu as pltpu, tpu_sc as plsc

NC = 2
NS = 16
LANES = 16

NP = 10240
ROWS_PER_TILE = NP // NS
CHUNK = 128
CHUNKS_PER_TILE = 40
EP = NC * NS * CHUNKS_PER_TILE * CHUNK

_MESH = plsc.VectorSubcoreMesh(core_axis_name="c", subcore_axis_name="s")


def _deg_body(dst_hbm, out_hbm, dst_v, deg_v, acc_v, tmp_v, shared):
    c = lax.axis_index("c")
    s = lax.axis_index("s")
    w = c * NS + s

    pltpu.sync_copy(dst_hbm.at[w], dst_v)

    zeros16 = jnp.zeros((LANES,), jnp.float32)

    def _zero(j, _):
        deg_v[pl.ds(j * LANES, LANES)] = zeros16
        return 0

    lax.fori_loop(0, NP // LANES, _zero, 0)

    ones16 = jnp.ones((LANES,), jnp.float32)

    def _edges(j, _):
        for k in range(CHUNK // LANES):
            d = dst_v[j, pl.ds(k * LANES, LANES)]
            plsc.addupdate_scatter(deg_v, [d], ones16)
        return 0

    lax.fori_loop(0, CHUNKS_PER_TILE, _edges, 0)

    pltpu.sync_copy(deg_v, shared.at[s])
    plsc.subcore_barrier()

    base = s * ROWS_PER_TILE

    def _zacc(j, _):
        acc_v[pl.ds(j * LANES, LANES)] = zeros16
        return 0

    lax.fori_loop(0, ROWS_PER_TILE // LANES, _zacc, 0)

    for t in range(NS):
        pltpu.sync_copy(shared.at[t, pl.ds(base, ROWS_PER_TILE)], tmp_v)

        def _acc(j, _):
            sl = pl.ds(j * LANES, LANES)
            acc_v[sl] = acc_v[sl] + tmp_v[sl]
            return 0

        lax.fori_loop(0, ROWS_PER_TILE // LANES, _acc, 0)

    pltpu.sync_copy(acc_v, out_hbm.at[c, pl.ds(base, ROWS_PER_TILE)])


_deg_kernel = pl.kernel(
    _deg_body,
    out_type=jax.ShapeDtypeStruct((NC, NP), jnp.float32),
    mesh=_MESH,
    scratch_types=[
        pltpu.VMEM((CHUNKS_PER_TILE, CHUNK), jnp.int32),
        pltpu.VMEM((NP,), jnp.float32),
        pltpu.VMEM((ROWS_PER_TILE,), jnp.float32),
        pltpu.VMEM((ROWS_PER_TILE,), jnp.float32),
        pltpu.VMEM_SHARED((NS, NP), jnp.float32),
    ],
    compiler_params=pltpu.CompilerParams(
        needs_layout_passes=False, use_tc_tiling_on_sc=False
    ),
)


_NBUF = 8


def _edge_body(g_hbm, src_hbm, dst_hbm, out_hbm, src_v, dst_v, rows_v, zbuf_v,
               shared, *sems, h):
    gsems = sems[:_NBUF]
    ssems = sems[_NBUF:]
    c = lax.axis_index("c")
    s = lax.axis_index("s")
    w = c * NS + s

    pltpu.sync_copy(src_hbm.at[w], src_v)
    pltpu.sync_copy(dst_hbm.at[w], dst_v)

    zeros16 = jnp.zeros((LANES,), jnp.float32)

    def _zero(j, _):
        for k in range(h // LANES):
            zbuf_v[j, pl.ds(k * LANES, LANES)] = zeros16
        return 0

    lax.fori_loop(0, CHUNK, _zero, 0)

    base = s * ROWS_PER_TILE
    for i in range(ROWS_PER_TILE // CHUNK):
        pltpu.sync_copy(zbuf_v, shared.at[pl.ds(base + i * CHUNK, CHUNK)])
    plsc.subcore_barrier()

    gd = [None] * _NBUF
    sd = [None] * _NBUF
    for b in range(_NBUF):
        gd[b] = pltpu.async_copy(g_hbm.at[src_v.at[b]], rows_v.at[b], gsems[b])
    for j in range(CHUNKS_PER_TILE):
        b = j % _NBUF
        gd[b].wait()
        sd[b] = pltpu.async_copy(
            rows_v.at[b], shared.at[dst_v.at[j]], ssems[b], add=True
        )
        if j + _NBUF < CHUNKS_PER_TILE:
            sd[b].wait()
            gd[b] = pltpu.async_copy(
                g_hbm.at[src_v.at[j + _NBUF]], rows_v.at[b], gsems[b]
            )
    for j in range(CHUNKS_PER_TILE - _NBUF, CHUNKS_PER_TILE):
        sd[j % _NBUF].wait()

    plsc.subcore_barrier()
    for i in range(ROWS_PER_TILE // CHUNK):
        pltpu.sync_copy(
            shared.at[pl.ds(base + i * CHUNK, CHUNK)],
            out_hbm.at[c, pl.ds(base + i * CHUNK, CHUNK)],
        )


@functools.cache
def _edge_kernel(h):
    return pl.kernel(
        functools.partial(_edge_body, h=h),
        out_type=jax.ShapeDtypeStruct((NC, NP, h), jnp.float32),
        mesh=_MESH,
        scratch_types=[
            pltpu.VMEM((CHUNKS_PER_TILE, CHUNK), jnp.int32),
            pltpu.VMEM((CHUNKS_PER_TILE, CHUNK), jnp.int32),
            pltpu.VMEM((_NBUF, CHUNK, h), jnp.float32),
            pltpu.VMEM((CHUNK, h), jnp.float32),
            pltpu.VMEM_SHARED((NP, h), jnp.float32),
        ]
        + [pltpu.SemaphoreType.DMA] * (2 * _NBUF),
        compiler_params=pltpu.CompilerParams(use_tc_tiling_on_sc=False),
    )


_BLK = 2048
_GRID = NP // _BLK


def _k1_body(deg_ref, x_ref, w_ref, h_ref, g_ref, dinv_ref, ood_ref):
    deg = jnp.sum(deg_ref[...], axis=0) + 1.0
    dinv = lax.rsqrt(deg)
    ood = 1.0 / deg
    hmat = jnp.dot(x_ref[...], w_ref[...], preferred_element_type=jnp.float32)
    h_ref[...] = hmat
    g_ref[...] = hmat * dinv
    dinv_ref[...] = dinv
    ood_ref[...] = ood


def _tc_first(deg2, xp, W1):
    h1w = W1.shape[1]
    return pl.pallas_call(
        _k1_body,
        grid=(_GRID,),
        in_specs=[
            pl.BlockSpec((NC, _BLK, 1), lambda i: (0, i, 0)),
            pl.BlockSpec((_BLK, xp.shape[1]), lambda i: (i, 0)),
            pl.BlockSpec(W1.shape, lambda i: (0, 0)),
        ],
        out_specs=[
            pl.BlockSpec((_BLK, h1w), lambda i: (i, 0)),
            pl.BlockSpec((_BLK, h1w), lambda i: (i, 0)),
            pl.BlockSpec((_BLK, 1), lambda i: (i, 0)),
            pl.BlockSpec((_BLK, 1), lambda i: (i, 0)),
        ],
        out_shape=[
            jax.ShapeDtypeStruct((NP, h1w), jnp.float32),
            jax.ShapeDtypeStruct((NP, h1w), jnp.float32),
            jax.ShapeDtypeStruct((NP, 1), jnp.float32),
            jax.ShapeDtypeStruct((NP, 1), jnp.float32),
        ],
    )(deg2, xp, W1)


def _k2_body(parts_ref, hcur_ref, dinv_ref, ood_ref, b_ref, w_ref,
             hn_ref, gn_ref):
    dinv = dinv_ref[...]
    z = (parts_ref[0] + parts_ref[1]) * dinv
    z = z + hcur_ref[...] * ood_ref[...] + b_ref[...]
    z = jnp.maximum(z, 0.0)
    hn = jnp.dot(z, w_ref[...], preferred_element_type=jnp.float32)
    hn_ref[...] = hn
    gn_ref[...] = hn * dinv


def _tc_mid(parts, hcur, dinv, ood, b, Wn):
    hw = hcur.shape[1]
    nw = Wn.shape[1]
    return pl.pallas_call(
        _k2_body,
        grid=(_GRID,),
        in_specs=[
            pl.BlockSpec((NC, _BLK, hw), lambda i: (0, i, 0)),
            pl.BlockSpec((_BLK, hw), lambda i: (i, 0)),
            pl.BlockSpec((_BLK, 1), lambda i: (i, 0)),
            pl.BlockSpec((_BLK, 1), lambda i: (i, 0)),
            pl.BlockSpec((1, hw), lambda i: (0, 0)),
            pl.BlockSpec((hw, nw), lambda i: (0, 0)),
        ],
        out_specs=[
            pl.BlockSpec((_BLK, nw), lambda i: (i, 0)),
            pl.BlockSpec((_BLK, nw), lambda i: (i, 0)),
        ],
        out_shape=[
            jax.ShapeDtypeStruct((NP, nw), jnp.float32),
            jax.ShapeDtypeStruct((NP, nw), jnp.float32),
        ],
    )(parts, hcur, dinv, ood, b, Wn)


def _k3_body(parts_ref, hcur_ref, dinv_ref, ood_ref, b_ref, out_ref, *, valid):
    logits = (parts_ref[0] + parts_ref[1]) * dinv_ref[...]
    logits = logits + hcur_ref[...] * ood_ref[...] + b_ref[...]
    cols = lax.broadcasted_iota(jnp.int32, logits.shape, 1)
    logits = jnp.where(cols < valid, logits, -1e30)
    m = jnp.max(logits, axis=-1, keepdims=True)
    e = jnp.exp(logits - m)
    out_ref[...] = e / jnp.sum(e, axis=-1, keepdims=True)


def _tc_last(parts, hcur, dinv, ood, b, valid):
    hw = hcur.shape[1]
    return pl.pallas_call(
        functools.partial(_k3_body, valid=valid),
        grid=(_GRID,),
        in_specs=[
            pl.BlockSpec((NC, _BLK, hw), lambda i: (0, i, 0)),
            pl.BlockSpec((_BLK, hw), lambda i: (i, 0)),
            pl.BlockSpec((_BLK, 1), lambda i: (i, 0)),
            pl.BlockSpec((_BLK, 1), lambda i: (i, 0)),
            pl.BlockSpec((1, hw), lambda i: (0, 0)),
        ],
        out_specs=pl.BlockSpec((_BLK, hw), lambda i: (i, 0)),
        out_shape=jax.ShapeDtypeStruct((NP, hw), jnp.float32),
    )(parts, hcur, dinv, ood, b)


def kernel(x, edge_index, W1, b1, W2, b2, W3, b3):
    n, _ = x.shape
    e = edge_index.shape[1]

    xp = jnp.pad(x, ((0, NP - n), (0, 0)))
    nw = NC * NS
    per_w = EP // nw
    pad_w = per_w - e // nw
    pad_idx = (
        n
        + (jnp.arange(nw * pad_w, dtype=jnp.int32) * 7) % (NP - n)
    ).reshape(nw, pad_w)
    src3 = jnp.concatenate(
        [edge_index[0].reshape(nw, e // nw), pad_idx], axis=1
    ).reshape(nw, CHUNKS_PER_TILE, CHUNK)
    dst3 = jnp.concatenate(
        [edge_index[1].reshape(nw, e // nw), pad_idx], axis=1
    ).reshape(nw, CHUNKS_PER_TILE, CHUNK)

    nclass = W3.shape[1]
    W3p = jnp.pad(W3, ((0, 0), (0, 48 - nclass)))
    b3p = jnp.pad(b3, (0, 48 - nclass))

    deg_parts = _deg_kernel(dst3)
    deg2 = deg_parts.reshape(NC, NP, 1)

    h1, g1, dinv, ood = _tc_first(deg2, xp, W1)
    parts1 = _edge_kernel(W1.shape[1])(g1, src3, dst3)
    h2, g2 = _tc_mid(parts1, h1, dinv, ood, b1.reshape(1, -1), W2)
    parts2 = _edge_kernel(W2.shape[1])(g2, src3, dst3)
    h3, g3 = _tc_mid(parts2, h2, dinv, ood, b2.reshape(1, -1), W3p)
    parts3 = _edge_kernel(48)(g3, src3, dst3)
    out = _tc_last(parts3, h3, dinv, ood, b3p.reshape(1, -1), nclass)
    return out[:n, :nclass]


# speedup vs baseline: 21.6832x; 1.0094x over previous
import functools

import numpy as np

import jax
import jax.numpy as jnp
from jax import lax
from jax.experimental import pallas as pl
from jax.experimental.pallas import tpu as pltpu, tpu_sc as plsc

NC = 2
NS = 16
LANES = 16

NP = 10240
ROWS_PER_TILE = NP // NS
CHUNK = 128
CHUNKS_PER_TILE = 40
EP = NC * NS * CHUNKS_PER_TILE * CHUNK

_MESH = plsc.VectorSubcoreMesh(core_axis_name="c", subcore_axis_name="s")


def _deg_body(dst_hbm, out_hbm, dst_v, deg_v, acc_v, tmp_v, shared):
    c = lax.axis_index("c")
    s = lax.axis_index("s")
    w = c * NS + s

    pltpu.sync_copy(dst_hbm.at[w], dst_v)

    zeros16 = jnp.zeros((LANES,), jnp.float32)

    def _zero(j, _):
        deg_v[pl.ds(j * LANES, LANES)] = zeros16
        return 0

    lax.fori_loop(0, NP // LANES, _zero, 0)

    ones16 = jnp.ones((LANES,), jnp.float32)

    def _edges(j, _):
        for k in range(CHUNK // LANES):
            d = dst_v[j, pl.ds(k * LANES, LANES)]
            plsc.addupdate_scatter(deg_v, [d], ones16)
        return 0

    lax.fori_loop(0, CHUNKS_PER_TILE, _edges, 0)

    pltpu.sync_copy(deg_v, shared.at[s])
    plsc.subcore_barrier()

    base = s * ROWS_PER_TILE

    def _zacc(j, _):
        acc_v[pl.ds(j * LANES, LANES)] = zeros16
        return 0

    lax.fori_loop(0, ROWS_PER_TILE // LANES, _zacc, 0)

    for t in range(NS):
        pltpu.sync_copy(shared.at[t, pl.ds(base, ROWS_PER_TILE)], tmp_v)

        def _acc(j, _):
            sl = pl.ds(j * LANES, LANES)
            acc_v[sl] = acc_v[sl] + tmp_v[sl]
            return 0

        lax.fori_loop(0, ROWS_PER_TILE // LANES, _acc, 0)

    pltpu.sync_copy(acc_v, out_hbm.at[c, pl.ds(base, ROWS_PER_TILE)])


_deg_kernel = pl.kernel(
    _deg_body,
    out_type=jax.ShapeDtypeStruct((NC, NP), jnp.float32),
    mesh=_MESH,
    scratch_types=[
        pltpu.VMEM((CHUNKS_PER_TILE, CHUNK), jnp.int32),
        pltpu.VMEM((NP,), jnp.float32),
        pltpu.VMEM((ROWS_PER_TILE,), jnp.float32),
        pltpu.VMEM((ROWS_PER_TILE,), jnp.float32),
        pltpu.VMEM_SHARED((NS, NP), jnp.float32),
    ],
    compiler_params=pltpu.CompilerParams(
        needs_layout_passes=False, use_tc_tiling_on_sc=False
    ),
)


_NBUF = 8


def _edge_body(g_hbm, src_hbm, dst_hbm, out_hbm, src_v, dst_v, rows_v, zbuf_v,
               shared, *sems, h):
    gsems = sems[:_NBUF]
    ssems = sems[_NBUF:]
    c = lax.axis_index("c")
    s = lax.axis_index("s")
    w = c * NS + s

    pltpu.sync_copy(src_hbm.at[w], src_v)
    pltpu.sync_copy(dst_hbm.at[w], dst_v)

    zeros16 = jnp.zeros((LANES,), jnp.float32)

    def _zero(j, _):
        for k in range(h // LANES):
            zbuf_v[j, pl.ds(k * LANES, LANES)] = zeros16
        return 0

    lax.fori_loop(0, CHUNK, _zero, 0)

    base = s * ROWS_PER_TILE
    for i in range(ROWS_PER_TILE // CHUNK):
        pltpu.sync_copy(zbuf_v, shared.at[pl.ds(base + i * CHUNK, CHUNK)])
    plsc.subcore_barrier()

    gd = [None] * _NBUF
    sd = [None] * _NBUF
    for b in range(_NBUF):
        gd[b] = pltpu.async_copy(g_hbm.at[src_v.at[b]], rows_v.at[b], gsems[b])
    for j in range(CHUNKS_PER_TILE):
        b = j % _NBUF
        gd[b].wait()
        sd[b] = pltpu.async_copy(
            rows_v.at[b], shared.at[dst_v.at[j]], ssems[b], add=True
        )
        if j + _NBUF < CHUNKS_PER_TILE:
            sd[b].wait()
            gd[b] = pltpu.async_copy(
                g_hbm.at[src_v.at[j + _NBUF]], rows_v.at[b], gsems[b]
            )
    for j in range(CHUNKS_PER_TILE - _NBUF, CHUNKS_PER_TILE):
        sd[j % _NBUF].wait()

    plsc.subcore_barrier()
    for i in range(ROWS_PER_TILE // CHUNK):
        pltpu.sync_copy(
            shared.at[pl.ds(base + i * CHUNK, CHUNK)],
            out_hbm.at[c, pl.ds(base + i * CHUNK, CHUNK)],
        )


@functools.cache
def _edge_kernel(h):
    return pl.kernel(
        functools.partial(_edge_body, h=h),
        out_type=jax.ShapeDtypeStruct((NC, NP, h), jnp.float32),
        mesh=_MESH,
        scratch_types=[
            pltpu.VMEM((CHUNKS_PER_TILE, CHUNK), jnp.int32),
            pltpu.VMEM((CHUNKS_PER_TILE, CHUNK), jnp.int32),
            pltpu.VMEM((_NBUF, CHUNK, h), jnp.float32),
            pltpu.VMEM((CHUNK, h), jnp.float32),
            pltpu.VMEM_SHARED((NP, h), jnp.float32),
        ]
        + [pltpu.SemaphoreType.DMA] * (2 * _NBUF),
        compiler_params=pltpu.CompilerParams(use_tc_tiling_on_sc=False),
    )


N_REAL = 10000
_BLK = 2000
_GRID = N_REAL // _BLK


def _k1_body(deg_ref, x_ref, w_ref, h_ref, g_ref, dinv_ref, ood_ref):
    deg = jnp.sum(deg_ref[...], axis=0) + 1.0
    dinv = lax.rsqrt(deg)
    ood = 1.0 / deg
    hmat = jnp.dot(x_ref[...], w_ref[...], preferred_element_type=jnp.float32)
    h_ref[...] = hmat
    g_ref[...] = hmat * dinv
    dinv_ref[...] = dinv
    ood_ref[...] = ood


def _tc_first(deg2, x, W1):
    h1w = W1.shape[1]
    return pl.pallas_call(
        _k1_body,
        grid=(_GRID,),
        in_specs=[
            pl.BlockSpec((NC, _BLK, 1), lambda i: (0, i, 0)),
            pl.BlockSpec((_BLK, x.shape[1]), lambda i: (i, 0)),
            pl.BlockSpec(W1.shape, lambda i: (0, 0)),
        ],
        out_specs=[
            pl.BlockSpec((_BLK, h1w), lambda i: (i, 0)),
            pl.BlockSpec((_BLK, h1w), lambda i: (i, 0)),
            pl.BlockSpec((_BLK, 1), lambda i: (i, 0)),
            pl.BlockSpec((_BLK, 1), lambda i: (i, 0)),
        ],
        out_shape=[
            jax.ShapeDtypeStruct((N_REAL, h1w), jnp.float32),
            jax.ShapeDtypeStruct((NP, h1w), jnp.float32),
            jax.ShapeDtypeStruct((N_REAL, 1), jnp.float32),
            jax.ShapeDtypeStruct((N_REAL, 1), jnp.float32),
        ],
    )(deg2, x, W1)


def _k2_body(parts_ref, hcur_ref, dinv_ref, ood_ref, b_ref, w_ref,
             hn_ref, gn_ref):
    dinv = dinv_ref[...]
    z = (parts_ref[0] + parts_ref[1]) * dinv
    z = z + hcur_ref[...] * ood_ref[...] + b_ref[...]
    z = jnp.maximum(z, 0.0)
    hn = jnp.dot(z, w_ref[...], preferred_element_type=jnp.float32)
    hn_ref[...] = hn
    nw = hn.shape[1]
    gw = gn_ref.shape[1]
    if gw == nw:
        gn_ref[...] = hn * dinv
    else:
        gn_ref[:, :nw] = hn * dinv
        gn_ref[:, nw:] = jnp.zeros((hn.shape[0], gw - nw), jnp.float32)


def _tc_mid(parts, hcur, dinv, ood, b, Wn, gw):
    hw = hcur.shape[1]
    nw = Wn.shape[1]
    return pl.pallas_call(
        _k2_body,
        grid=(_GRID,),
        in_specs=[
            pl.BlockSpec((NC, _BLK, hw), lambda i: (0, i, 0)),
            pl.BlockSpec((_BLK, hw), lambda i: (i, 0)),
            pl.BlockSpec((_BLK, 1), lambda i: (i, 0)),
            pl.BlockSpec((_BLK, 1), lambda i: (i, 0)),
            pl.BlockSpec((1, hw), lambda i: (0, 0)),
            pl.BlockSpec((hw, nw), lambda i: (0, 0)),
        ],
        out_specs=[
            pl.BlockSpec((_BLK, nw), lambda i: (i, 0)),
            pl.BlockSpec((_BLK, gw), lambda i: (i, 0)),
        ],
        out_shape=[
            jax.ShapeDtypeStruct((N_REAL, nw), jnp.float32),
            jax.ShapeDtypeStruct((NP, gw), jnp.float32),
        ],
    )(parts, hcur, dinv, ood, b, Wn)


def _k3_body(parts_ref, hcur_ref, dinv_ref, ood_ref, b_ref, out_ref):
    nw = out_ref.shape[1]
    logits = (parts_ref[0][:, :nw] + parts_ref[1][:, :nw]) * dinv_ref[...]
    logits = logits + hcur_ref[...] * ood_ref[...] + b_ref[...]
    m = jnp.max(logits, axis=-1, keepdims=True)
    e = jnp.exp(logits - m)
    out_ref[...] = e / jnp.sum(e, axis=-1, keepdims=True)


def _tc_last(parts, hcur, dinv, ood, b):
    gw = parts.shape[2]
    nw = hcur.shape[1]
    return pl.pallas_call(
        _k3_body,
        grid=(_GRID,),
        in_specs=[
            pl.BlockSpec((NC, _BLK, gw), lambda i: (0, i, 0)),
            pl.BlockSpec((_BLK, nw), lambda i: (i, 0)),
            pl.BlockSpec((_BLK, 1), lambda i: (i, 0)),
            pl.BlockSpec((_BLK, 1), lambda i: (i, 0)),
            pl.BlockSpec((1, nw), lambda i: (0, 0)),
        ],
        out_specs=pl.BlockSpec((_BLK, nw), lambda i: (i, 0)),
        out_shape=jax.ShapeDtypeStruct((N_REAL, nw), jnp.float32),
    )(parts, hcur, dinv, ood, b)


def kernel(x, edge_index, W1, b1, W2, b2, W3, b3):
    n, _ = x.shape
    e = edge_index.shape[1]

    nw = NC * NS
    per_w = EP // nw
    pad_w = per_w - e // nw
    pad_idx = jnp.asarray(
        n + (np.arange(nw * pad_w, dtype=np.int32) * 7) % (NP - n),
        dtype=jnp.int32,
    ).reshape(nw, pad_w)
    src3 = jnp.concatenate(
        [edge_index[0].reshape(nw, e // nw), pad_idx], axis=1
    ).reshape(nw, CHUNKS_PER_TILE, CHUNK)
    dst3 = jnp.concatenate(
        [edge_index[1].reshape(nw, e // nw), pad_idx], axis=1
    ).reshape(nw, CHUNKS_PER_TILE, CHUNK)

    deg_parts = _deg_kernel(dst3)
    deg2 = deg_parts.reshape(NC, NP, 1)

    h1, g1, dinv, ood = _tc_first(deg2, x, W1)
    parts1 = _edge_kernel(W1.shape[1])(g1, src3, dst3)
    h2, g2 = _tc_mid(parts1, h1, dinv, ood, b1.reshape(1, -1), W2, 64)
    parts2 = _edge_kernel(64)(g2, src3, dst3)
    h3, g3 = _tc_mid(parts2, h2, dinv, ood, b2.reshape(1, -1), W3, 48)
    parts3 = _edge_kernel(48)(g3, src3, dst3)
    return _tc_last(parts3, h3, dinv, ood, b3.reshape(1, -1))


# speedup vs baseline: 21.9266x; 1.0112x over previous
import functools

import numpy as np

import jax
import jax.numpy as jnp
from jax import lax
from jax.experimental import pallas as pl
from jax.experimental.pallas import tpu as pltpu, tpu_sc as plsc

NC = 2
NS = 16
LANES = 16

NP = 10240
ROWS_PER_TILE = NP // NS
CHUNK = 128
CHUNKS_PER_TILE = 40
EP = NC * NS * CHUNKS_PER_TILE * CHUNK

_MESH = plsc.VectorSubcoreMesh(core_axis_name="c", subcore_axis_name="s")


def _deg_body(dst_hbm, out_hbm, dst_v, deg_v, acc_v, tmp_v, shared):
    c = lax.axis_index("c")
    s = lax.axis_index("s")
    w = c * NS + s

    pltpu.sync_copy(dst_hbm.at[w], dst_v)

    zeros16 = jnp.zeros((LANES,), jnp.float32)

    def _zero(j, _):
        deg_v[pl.ds(j * LANES, LANES)] = zeros16
        return 0

    lax.fori_loop(0, NP // LANES, _zero, 0)

    ones16 = jnp.ones((LANES,), jnp.float32)

    def _edges(j, _):
        for k in range(CHUNK // LANES):
            d = dst_v[j, pl.ds(k * LANES, LANES)]
            plsc.addupdate_scatter(deg_v, [d], ones16)
        return 0

    lax.fori_loop(0, CHUNKS_PER_TILE, _edges, 0)

    pltpu.sync_copy(deg_v, shared.at[s])
    plsc.subcore_barrier()

    base = s * ROWS_PER_TILE

    def _zacc(j, _):
        acc_v[pl.ds(j * LANES, LANES)] = zeros16
        return 0

    lax.fori_loop(0, ROWS_PER_TILE // LANES, _zacc, 0)

    for t in range(NS):
        pltpu.sync_copy(shared.at[t, pl.ds(base, ROWS_PER_TILE)], tmp_v)

        def _acc(j, _):
            sl = pl.ds(j * LANES, LANES)
            acc_v[sl] = acc_v[sl] + tmp_v[sl]
            return 0

        lax.fori_loop(0, ROWS_PER_TILE // LANES, _acc, 0)

    pltpu.sync_copy(acc_v, out_hbm.at[c, pl.ds(base, ROWS_PER_TILE)])


_deg_kernel = pl.kernel(
    _deg_body,
    out_type=jax.ShapeDtypeStruct((NC, NP), jnp.float32),
    mesh=_MESH,
    scratch_types=[
        pltpu.VMEM((CHUNKS_PER_TILE, CHUNK), jnp.int32),
        pltpu.VMEM((NP,), jnp.float32),
        pltpu.VMEM((ROWS_PER_TILE,), jnp.float32),
        pltpu.VMEM((ROWS_PER_TILE,), jnp.float32),
        pltpu.VMEM_SHARED((NS, NP), jnp.float32),
    ],
    compiler_params=pltpu.CompilerParams(
        needs_layout_passes=False, use_tc_tiling_on_sc=False
    ),
)


_NBUF = 8


def _edge_body(g_hbm, src_hbm, dst_hbm, out_hbm, src_v, dst_v, rows_v, zbuf_v,
               shared, *sems, h):
    gsems = sems[:_NBUF]
    ssems = sems[_NBUF:]
    c = lax.axis_index("c")
    s = lax.axis_index("s")
    w = c * NS + s

    pltpu.sync_copy(src_hbm.at[w], src_v)
    pltpu.sync_copy(dst_hbm.at[w], dst_v)

    zeros16 = jnp.zeros((LANES,), jnp.float32)

    def _zero(j, _):
        for k in range(h // LANES):
            zbuf_v[j, pl.ds(k * LANES, LANES)] = zeros16
        return 0

    lax.fori_loop(0, CHUNK, _zero, 0)

    base = s * ROWS_PER_TILE
    for i in range(ROWS_PER_TILE // CHUNK):
        pltpu.sync_copy(zbuf_v, shared.at[pl.ds(base + i * CHUNK, CHUNK)])
    plsc.subcore_barrier()

    gd = [None] * _NBUF
    sd = [None] * _NBUF
    for b in range(_NBUF):
        gd[b] = pltpu.async_copy(g_hbm.at[src_v.at[b]], rows_v.at[b], gsems[b])
    for j in range(CHUNKS_PER_TILE):
        b = j % _NBUF
        gd[b].wait()
        sd[b] = pltpu.async_copy(
            rows_v.at[b], shared.at[dst_v.at[j]], ssems[b], add=True
        )
        if j + _NBUF < CHUNKS_PER_TILE:
            sd[b].wait()
            gd[b] = pltpu.async_copy(
                g_hbm.at[src_v.at[j + _NBUF]], rows_v.at[b], gsems[b]
            )
    for j in range(CHUNKS_PER_TILE - _NBUF, CHUNKS_PER_TILE):
        sd[j % _NBUF].wait()

    plsc.subcore_barrier()
    for i in range(ROWS_PER_TILE // CHUNK):
        pltpu.sync_copy(
            shared.at[pl.ds(base + i * CHUNK, CHUNK)],
            out_hbm.at[c, pl.ds(base + i * CHUNK, CHUNK)],
        )


@functools.cache
def _edge_kernel(h):
    return pl.kernel(
        functools.partial(_edge_body, h=h),
        out_type=jax.ShapeDtypeStruct((NC, NP, h), jnp.float32),
        mesh=_MESH,
        scratch_types=[
            pltpu.VMEM((CHUNKS_PER_TILE, CHUNK), jnp.int32),
            pltpu.VMEM((CHUNKS_PER_TILE, CHUNK), jnp.int32),
            pltpu.VMEM((_NBUF, CHUNK, h), jnp.float32),
            pltpu.VMEM((CHUNK, h), jnp.float32),
            pltpu.VMEM_SHARED((NP, h), jnp.float32),
        ]
        + [pltpu.SemaphoreType.DMA] * (2 * _NBUF),
        compiler_params=pltpu.CompilerParams(use_tc_tiling_on_sc=False),
    )


N_REAL = 10000
_BLK = 5000
_GRID = N_REAL // _BLK


def _k1_body(deg_ref, x_ref, w_ref, h_ref, g_ref, dinv_ref, ood_ref):
    deg = jnp.sum(deg_ref[...], axis=0) + 1.0
    dinv = lax.rsqrt(deg)
    ood = 1.0 / deg
    hmat = jnp.dot(x_ref[...], w_ref[...], preferred_element_type=jnp.float32)
    h_ref[...] = hmat
    g_ref[...] = hmat * dinv
    dinv_ref[...] = dinv
    ood_ref[...] = ood


def _tc_first(deg2, x, W1):
    h1w = W1.shape[1]
    return pl.pallas_call(
        _k1_body,
        grid=(_GRID,),
        in_specs=[
            pl.BlockSpec((NC, _BLK, 1), lambda i: (0, i, 0)),
            pl.BlockSpec((_BLK, x.shape[1]), lambda i: (i, 0)),
            pl.BlockSpec(W1.shape, lambda i: (0, 0)),
        ],
        out_specs=[
            pl.BlockSpec((_BLK, h1w), lambda i: (i, 0)),
            pl.BlockSpec((_BLK, h1w), lambda i: (i, 0)),
            pl.BlockSpec((_BLK, 1), lambda i: (i, 0)),
            pl.BlockSpec((_BLK, 1), lambda i: (i, 0)),
        ],
        out_shape=[
            jax.ShapeDtypeStruct((N_REAL, h1w), jnp.float32),
            jax.ShapeDtypeStruct((NP, h1w), jnp.float32),
            jax.ShapeDtypeStruct((N_REAL, 1), jnp.float32),
            jax.ShapeDtypeStruct((N_REAL, 1), jnp.float32),
        ],
    )(deg2, x, W1)


def _k2_body(parts_ref, hcur_ref, dinv_ref, ood_ref, b_ref, w_ref,
             hn_ref, gn_ref):
    dinv = dinv_ref[...]
    z = (parts_ref[0] + parts_ref[1]) * dinv
    z = z + hcur_ref[...] * ood_ref[...] + b_ref[...]
    z = jnp.maximum(z, 0.0)
    hn = jnp.dot(z, w_ref[...], preferred_element_type=jnp.float32)
    hn_ref[...] = hn
    nw = hn.shape[1]
    gw = gn_ref.shape[1]
    if gw == nw:
        gn_ref[...] = hn * dinv
    else:
        gn_ref[:, :nw] = hn * dinv
        gn_ref[:, nw:] = jnp.zeros((hn.shape[0], gw - nw), jnp.float32)


def _tc_mid(parts, hcur, dinv, ood, b, Wn, gw):
    hw = hcur.shape[1]
    nw = Wn.shape[1]
    return pl.pallas_call(
        _k2_body,
        grid=(_GRID,),
        in_specs=[
            pl.BlockSpec((NC, _BLK, hw), lambda i: (0, i, 0)),
            pl.BlockSpec((_BLK, hw), lambda i: (i, 0)),
            pl.BlockSpec((_BLK, 1), lambda i: (i, 0)),
            pl.BlockSpec((_BLK, 1), lambda i: (i, 0)),
            pl.BlockSpec((1, hw), lambda i: (0, 0)),
            pl.BlockSpec((hw, nw), lambda i: (0, 0)),
        ],
        out_specs=[
            pl.BlockSpec((_BLK, nw), lambda i: (i, 0)),
            pl.BlockSpec((_BLK, gw), lambda i: (i, 0)),
        ],
        out_shape=[
            jax.ShapeDtypeStruct((N_REAL, nw), jnp.float32),
            jax.ShapeDtypeStruct((NP, gw), jnp.float32),
        ],
    )(parts, hcur, dinv, ood, b, Wn)


def _k3_body(parts_ref, hcur_ref, dinv_ref, ood_ref, b_ref, out_ref):
    nw = out_ref.shape[1]
    logits = (parts_ref[0][:, :nw] + parts_ref[1][:, :nw]) * dinv_ref[...]
    logits = logits + hcur_ref[...] * ood_ref[...] + b_ref[...]
    m = jnp.max(logits, axis=-1, keepdims=True)
    e = jnp.exp(logits - m)
    out_ref[...] = e / jnp.sum(e, axis=-1, keepdims=True)


def _tc_last(parts, hcur, dinv, ood, b):
    gw = parts.shape[2]
    nw = hcur.shape[1]
    return pl.pallas_call(
        _k3_body,
        grid=(_GRID,),
        in_specs=[
            pl.BlockSpec((NC, _BLK, gw), lambda i: (0, i, 0)),
            pl.BlockSpec((_BLK, nw), lambda i: (i, 0)),
            pl.BlockSpec((_BLK, 1), lambda i: (i, 0)),
            pl.BlockSpec((_BLK, 1), lambda i: (i, 0)),
            pl.BlockSpec((1, nw), lambda i: (0, 0)),
        ],
        out_specs=pl.BlockSpec((_BLK, nw), lambda i: (i, 0)),
        out_shape=jax.ShapeDtypeStruct((N_REAL, nw), jnp.float32),
    )(parts, hcur, dinv, ood, b)


def kernel(x, edge_index, W1, b1, W2, b2, W3, b3):
    n, _ = x.shape
    e = edge_index.shape[1]

    nw = NC * NS
    per_w = EP // nw
    pad_w = per_w - e // nw
    pad_idx = jnp.asarray(
        n + (np.arange(nw * pad_w, dtype=np.int32) * 7) % (NP - n),
        dtype=jnp.int32,
    ).reshape(nw, pad_w)
    src3 = jnp.concatenate(
        [edge_index[0].reshape(nw, e // nw), pad_idx], axis=1
    ).reshape(nw, CHUNKS_PER_TILE, CHUNK)
    dst3 = jnp.concatenate(
        [edge_index[1].reshape(nw, e // nw), pad_idx], axis=1
    ).reshape(nw, CHUNKS_PER_TILE, CHUNK)

    deg_parts = _deg_kernel(dst3)
    deg2 = deg_parts.reshape(NC, NP, 1)

    h1, g1, dinv, ood = _tc_first(deg2, x, W1)
    parts1 = _edge_kernel(W1.shape[1])(g1, src3, dst3)
    h2, g2 = _tc_mid(parts1, h1, dinv, ood, b1.reshape(1, -1), W2, 64)
    parts2 = _edge_kernel(64)(g2, src3, dst3)
    h3, g3 = _tc_mid(parts2, h2, dinv, ood, b2.reshape(1, -1), W3, 48)
    parts3 = _edge_kernel(48)(g3, src3, dst3)
    return _tc_last(parts3, h3, dinv, ood, b3.reshape(1, -1))


# speedup vs baseline: 23.5206x; 1.0727x over previous
import functools

import numpy as np

import jax
import jax.numpy as jnp
from jax import lax
from jax.experimental import pallas as pl
from jax.experimental.pallas import tpu as pltpu, tpu_sc as plsc

NC = 2
NS = 16
LANES = 16

NP = 10240
ROWS_PER_TILE = NP // NS
CHUNK = 128
CHUNKS_PER_TILE = 40
EP = NC * NS * CHUNKS_PER_TILE * CHUNK

_MESH = plsc.VectorSubcoreMesh(core_axis_name="c", subcore_axis_name="s")


def _deg_body(dst_hbm, out_hbm, dst_v, deg_v, acc_v, tmp_v, shared):
    c = lax.axis_index("c")
    s = lax.axis_index("s")
    w = c * NS + s

    pltpu.sync_copy(dst_hbm.at[w], dst_v)

    zeros16 = jnp.zeros((LANES,), jnp.float32)

    def _zero(j, _):
        deg_v[pl.ds(j * LANES, LANES)] = zeros16
        return 0

    lax.fori_loop(0, NP // LANES, _zero, 0)

    ones16 = jnp.ones((LANES,), jnp.float32)

    def _edges(j, _):
        for k in range(CHUNK // LANES):
            d = dst_v[j, pl.ds(k * LANES, LANES)]
            plsc.addupdate_scatter(deg_v, [d], ones16)
        return 0

    lax.fori_loop(0, CHUNKS_PER_TILE, _edges, 0)

    pltpu.sync_copy(deg_v, shared.at[s])
    plsc.subcore_barrier()

    base = s * ROWS_PER_TILE

    def _zacc(j, _):
        acc_v[pl.ds(j * LANES, LANES)] = zeros16
        return 0

    lax.fori_loop(0, ROWS_PER_TILE // LANES, _zacc, 0)

    for t in range(NS):
        pltpu.sync_copy(shared.at[t, pl.ds(base, ROWS_PER_TILE)], tmp_v)

        def _acc(j, _):
            sl = pl.ds(j * LANES, LANES)
            acc_v[sl] = acc_v[sl] + tmp_v[sl]
            return 0

        lax.fori_loop(0, ROWS_PER_TILE // LANES, _acc, 0)

    pltpu.sync_copy(acc_v, out_hbm.at[c, pl.ds(base, ROWS_PER_TILE)])


_deg_kernel = pl.kernel(
    _deg_body,
    out_type=jax.ShapeDtypeStruct((NC, NP), jnp.float32),
    mesh=_MESH,
    scratch_types=[
        pltpu.VMEM((CHUNKS_PER_TILE, CHUNK), jnp.int32),
        pltpu.VMEM((NP,), jnp.float32),
        pltpu.VMEM((ROWS_PER_TILE,), jnp.float32),
        pltpu.VMEM((ROWS_PER_TILE,), jnp.float32),
        pltpu.VMEM_SHARED((NS, NP), jnp.float32),
    ],
    compiler_params=pltpu.CompilerParams(
        needs_layout_passes=False, use_tc_tiling_on_sc=False
    ),
)


_NBUF = 8


def _edge_body(g_hbm, src_hbm, dst_hbm, out_hbm, src_v, dst_v, rows_v, zbuf_v,
               shared, *sems, h):
    gsems = sems[:_NBUF]
    ssems = sems[_NBUF:]
    c = lax.axis_index("c")
    s = lax.axis_index("s")
    w = c * NS + s

    pltpu.sync_copy(src_hbm.at[w], src_v)
    pltpu.sync_copy(dst_hbm.at[w], dst_v)

    zeros16 = jnp.zeros((LANES,), jnp.float32)

    def _zero(j, _):
        for k in range(h // LANES):
            zbuf_v[j, pl.ds(k * LANES, LANES)] = zeros16
        return 0

    lax.fori_loop(0, CHUNK, _zero, 0)

    base = s * ROWS_PER_TILE
    for i in range(ROWS_PER_TILE // CHUNK):
        pltpu.sync_copy(zbuf_v, shared.at[pl.ds(base + i * CHUNK, CHUNK)])
    plsc.subcore_barrier()

    gd = [None] * _NBUF
    sd = [None] * _NBUF
    for b in range(_NBUF):
        gd[b] = pltpu.async_copy(g_hbm.at[src_v.at[b]], rows_v.at[b], gsems[b])
    for j in range(CHUNKS_PER_TILE):
        b = j % _NBUF
        gd[b].wait()
        sd[b] = pltpu.async_copy(
            rows_v.at[b], shared.at[dst_v.at[j]], ssems[b], add=True
        )
        if j + _NBUF < CHUNKS_PER_TILE:
            sd[b].wait()
            gd[b] = pltpu.async_copy(
                g_hbm.at[src_v.at[j + _NBUF]], rows_v.at[b], gsems[b]
            )
    for j in range(CHUNKS_PER_TILE - _NBUF, CHUNKS_PER_TILE):
        sd[j % _NBUF].wait()

    plsc.subcore_barrier()
    for i in range(ROWS_PER_TILE // CHUNK):
        pltpu.sync_copy(
            shared.at[pl.ds(base + i * CHUNK, CHUNK)],
            out_hbm.at[c, pl.ds(base + i * CHUNK, CHUNK)],
        )


@functools.cache
def _edge_kernel(h):
    return pl.kernel(
        functools.partial(_edge_body, h=h),
        out_type=jax.ShapeDtypeStruct((NC, NP, h), jnp.float32),
        mesh=_MESH,
        scratch_types=[
            pltpu.VMEM((CHUNKS_PER_TILE, CHUNK), jnp.int32),
            pltpu.VMEM((CHUNKS_PER_TILE, CHUNK), jnp.int32),
            pltpu.VMEM((_NBUF, CHUNK, h), jnp.float32),
            pltpu.VMEM((CHUNK, h), jnp.float32),
            pltpu.VMEM_SHARED((NP, h), jnp.float32),
        ]
        + [pltpu.SemaphoreType.DMA] * (2 * _NBUF),
        compiler_params=pltpu.CompilerParams(use_tc_tiling_on_sc=False),
    )


N_REAL = 10000


def _k1_body(deg_ref, x_ref, w_ref, h_ref, g_ref, dinv_ref, ood_ref):
    deg = deg_ref[0] + deg_ref[1] + 1.0
    dinv = lax.rsqrt(deg)
    ood = 1.0 / deg
    dinv_ref[...] = dinv
    ood_ref[...] = ood
    dcol = dinv.reshape(NP, 1)[:N_REAL]
    hmat = jnp.dot(x_ref[...], w_ref[...], preferred_element_type=jnp.float32)
    h_ref[...] = hmat
    g_ref[...] = hmat * dcol


def _tc_first(deg_parts, x, W1):
    h1w = W1.shape[1]
    return pl.pallas_call(
        _k1_body,
        grid=(1,),
        in_specs=[
            pl.BlockSpec((NC, NP), lambda i: (0, 0)),
            pl.BlockSpec((N_REAL, x.shape[1]), lambda i: (0, 0)),
            pl.BlockSpec(W1.shape, lambda i: (0, 0)),
        ],
        out_specs=[
            pl.BlockSpec((N_REAL, h1w), lambda i: (0, 0)),
            pl.BlockSpec((N_REAL, h1w), lambda i: (0, 0)),
            pl.BlockSpec((NP,), lambda i: (0,)),
            pl.BlockSpec((NP,), lambda i: (0,)),
        ],
        out_shape=[
            jax.ShapeDtypeStruct((N_REAL, h1w), jnp.float32),
            jax.ShapeDtypeStruct((NP, h1w), jnp.float32),
            jax.ShapeDtypeStruct((NP,), jnp.float32),
            jax.ShapeDtypeStruct((NP,), jnp.float32),
        ],
    )(deg_parts, x, W1)


def _k2_body(parts_ref, hcur_ref, dinv_ref, ood_ref, b_ref, w_ref,
             hn_ref, gn_ref):
    dcol = dinv_ref[...].reshape(NP, 1)[:N_REAL]
    ocol = ood_ref[...].reshape(NP, 1)[:N_REAL]
    z = (parts_ref[0, :N_REAL] + parts_ref[1, :N_REAL]) * dcol
    z = z + hcur_ref[...] * ocol + b_ref[...]
    z = jnp.maximum(z, 0.0)
    hn = jnp.dot(z, w_ref[...], preferred_element_type=jnp.float32)
    hn_ref[...] = hn
    nw = hn.shape[1]
    gw = gn_ref.shape[1]
    if gw == nw:
        gn_ref[...] = hn * dcol
    else:
        gn_ref[:, :nw] = hn * dcol
        gn_ref[:, nw:] = jnp.zeros((hn.shape[0], gw - nw), jnp.float32)


def _tc_mid(parts, hcur, dinv, ood, b, Wn, gw):
    hw = hcur.shape[1]
    nw = Wn.shape[1]
    return pl.pallas_call(
        _k2_body,
        grid=(1,),
        in_specs=[
            pl.BlockSpec((NC, NP, hw), lambda i: (0, 0, 0)),
            pl.BlockSpec((N_REAL, hw), lambda i: (0, 0)),
            pl.BlockSpec((NP,), lambda i: (0,)),
            pl.BlockSpec((NP,), lambda i: (0,)),
            pl.BlockSpec((1, hw), lambda i: (0, 0)),
            pl.BlockSpec((hw, nw), lambda i: (0, 0)),
        ],
        out_specs=[
            pl.BlockSpec((N_REAL, nw), lambda i: (0, 0)),
            pl.BlockSpec((N_REAL, gw), lambda i: (0, 0)),
        ],
        out_shape=[
            jax.ShapeDtypeStruct((N_REAL, nw), jnp.float32),
            jax.ShapeDtypeStruct((NP, gw), jnp.float32),
        ],
    )(parts, hcur, dinv, ood, b, Wn)


def _k3_body(parts_ref, hcur_ref, dinv_ref, ood_ref, b_ref, out_ref):
    nw = out_ref.shape[1]
    dcol = dinv_ref[...].reshape(NP, 1)[:N_REAL]
    ocol = ood_ref[...].reshape(NP, 1)[:N_REAL]
    logits = (parts_ref[0, :N_REAL, :nw] + parts_ref[1, :N_REAL, :nw]) * dcol
    logits = logits + hcur_ref[...] * ocol + b_ref[...]
    m = jnp.max(logits, axis=-1, keepdims=True)
    e = jnp.exp(logits - m)
    out_ref[...] = e / jnp.sum(e, axis=-1, keepdims=True)


def _tc_last(parts, hcur, dinv, ood, b):
    gw = parts.shape[2]
    nw = hcur.shape[1]
    return pl.pallas_call(
        _k3_body,
        grid=(1,),
        in_specs=[
            pl.BlockSpec((NC, NP, gw), lambda i: (0, 0, 0)),
            pl.BlockSpec((N_REAL, nw), lambda i: (0, 0)),
            pl.BlockSpec((NP,), lambda i: (0,)),
            pl.BlockSpec((NP,), lambda i: (0,)),
            pl.BlockSpec((1, nw), lambda i: (0, 0)),
        ],
        out_specs=pl.BlockSpec((N_REAL, nw), lambda i: (0, 0)),
        out_shape=jax.ShapeDtypeStruct((N_REAL, nw), jnp.float32),
    )(parts, hcur, dinv, ood, b)


def kernel(x, edge_index, W1, b1, W2, b2, W3, b3):
    n, _ = x.shape
    e = edge_index.shape[1]

    nw = NC * NS
    per_w = EP // nw
    pad_w = per_w - e // nw
    pad_idx = jnp.asarray(
        n + (np.arange(nw * pad_w, dtype=np.int32) * 7) % (NP - n),
        dtype=jnp.int32,
    ).reshape(nw, pad_w)
    src3 = jnp.concatenate(
        [edge_index[0].reshape(nw, e // nw), pad_idx], axis=1
    ).reshape(nw, CHUNKS_PER_TILE, CHUNK)
    dst3 = jnp.concatenate(
        [edge_index[1].reshape(nw, e // nw), pad_idx], axis=1
    ).reshape(nw, CHUNKS_PER_TILE, CHUNK)

    deg_parts = _deg_kernel(dst3)

    h1, g1, dinv, ood = _tc_first(deg_parts, x, W1)
    parts1 = _edge_kernel(W1.shape[1])(g1, src3, dst3)
    h2, g2 = _tc_mid(parts1, h1, dinv, ood, b1.reshape(1, -1), W2, 64)
    parts2 = _edge_kernel(64)(g2, src3, dst3)
    h3, g3 = _tc_mid(parts2, h2, dinv, ood, b2.reshape(1, -1), W3, 48)
    parts3 = _edge_kernel(48)(g3, src3, dst3)
    return _tc_last(parts3, h3, dinv, ood, b3.reshape(1, -1))


# speedup vs baseline: 23.7767x; 1.0109x over previous
import functools

import numpy as np

import jax
import jax.numpy as jnp
from jax import lax
from jax.experimental import pallas as pl
from jax.experimental.pallas import tpu as pltpu, tpu_sc as plsc

NC = 2
NS = 16
LANES = 16

NP = 10240
ROWS_PER_TILE = NP // NS
CHUNK = 128
CHUNKS_PER_TILE = 40
EP = NC * NS * CHUNKS_PER_TILE * CHUNK

_MESH = plsc.VectorSubcoreMesh(core_axis_name="c", subcore_axis_name="s")


def _deg_body(dst_hbm, out_hbm, dst_v, deg_v, acc_v, tmp_v, shared):
    c = lax.axis_index("c")
    s = lax.axis_index("s")
    w = c * NS + s

    pltpu.sync_copy(dst_hbm.at[w], dst_v)

    zeros16 = jnp.zeros((LANES,), jnp.float32)

    def _zero(j, _):
        deg_v[pl.ds(j * LANES, LANES)] = zeros16
        return 0

    lax.fori_loop(0, NP // LANES, _zero, 0)

    ones16 = jnp.ones((LANES,), jnp.float32)

    def _edges(j, _):
        for k in range(CHUNK // LANES):
            d = dst_v[j, pl.ds(k * LANES, LANES)]
            plsc.addupdate_scatter(deg_v, [d], ones16)
        return 0

    lax.fori_loop(0, CHUNKS_PER_TILE, _edges, 0)

    pltpu.sync_copy(deg_v, shared.at[s])
    plsc.subcore_barrier()

    base = s * ROWS_PER_TILE

    def _zacc(j, _):
        acc_v[pl.ds(j * LANES, LANES)] = zeros16
        return 0

    lax.fori_loop(0, ROWS_PER_TILE // LANES, _zacc, 0)

    for t in range(NS):
        pltpu.sync_copy(shared.at[t, pl.ds(base, ROWS_PER_TILE)], tmp_v)

        def _acc(j, _):
            sl = pl.ds(j * LANES, LANES)
            acc_v[sl] = acc_v[sl] + tmp_v[sl]
            return 0

        lax.fori_loop(0, ROWS_PER_TILE // LANES, _acc, 0)

    pltpu.sync_copy(acc_v, out_hbm.at[c, pl.ds(base, ROWS_PER_TILE)])


_deg_kernel = pl.kernel(
    _deg_body,
    out_type=jax.ShapeDtypeStruct((NC, NP), jnp.float32),
    mesh=_MESH,
    scratch_types=[
        pltpu.VMEM((CHUNKS_PER_TILE, CHUNK), jnp.int32),
        pltpu.VMEM((NP,), jnp.float32),
        pltpu.VMEM((ROWS_PER_TILE,), jnp.float32),
        pltpu.VMEM((ROWS_PER_TILE,), jnp.float32),
        pltpu.VMEM_SHARED((NS, NP), jnp.float32),
    ],
    compiler_params=pltpu.CompilerParams(
        needs_layout_passes=False, use_tc_tiling_on_sc=False
    ),
)


_NBUF = 8


def _edge_body(g_hbm, src_hbm, dst_hbm, out_hbm, src_v, dst_v, rows_v, zbuf_v,
               shared, *sems, h):
    gsems = sems[:_NBUF]
    ssems = sems[_NBUF:]
    c = lax.axis_index("c")
    s = lax.axis_index("s")
    w = c * NS + s

    pltpu.sync_copy(src_hbm.at[w], src_v)
    pltpu.sync_copy(dst_hbm.at[w], dst_v)

    zeros16 = jnp.zeros((LANES,), jnp.float32)

    def _zero(j, _):
        for k in range(h // LANES):
            zbuf_v[j, pl.ds(k * LANES, LANES)] = zeros16
        return 0

    lax.fori_loop(0, CHUNK, _zero, 0)

    base = s * ROWS_PER_TILE
    for i in range(ROWS_PER_TILE // CHUNK):
        pltpu.sync_copy(zbuf_v, shared.at[pl.ds(base + i * CHUNK, CHUNK)])
    plsc.subcore_barrier()

    gd = [None] * _NBUF
    sd = [None] * _NBUF
    for b in range(_NBUF):
        gd[b] = pltpu.async_copy(g_hbm.at[src_v.at[b]], rows_v.at[b], gsems[b])
    for j in range(CHUNKS_PER_TILE):
        b = j % _NBUF
        gd[b].wait()
        sd[b] = pltpu.async_copy(
            rows_v.at[b], shared.at[dst_v.at[j]], ssems[b], add=True
        )
        if j + _NBUF < CHUNKS_PER_TILE:
            sd[b].wait()
            gd[b] = pltpu.async_copy(
                g_hbm.at[src_v.at[j + _NBUF]], rows_v.at[b], gsems[b]
            )
    for j in range(CHUNKS_PER_TILE - _NBUF, CHUNKS_PER_TILE):
        sd[j % _NBUF].wait()

    plsc.subcore_barrier()
    for i in range(ROWS_PER_TILE // CHUNK):
        pltpu.sync_copy(
            shared.at[pl.ds(base + i * CHUNK, CHUNK)],
            out_hbm.at[c, pl.ds(base + i * CHUNK, CHUNK)],
        )


@functools.cache
def _edge_kernel(h):
    return pl.kernel(
        functools.partial(_edge_body, h=h),
        out_type=jax.ShapeDtypeStruct((NC, NP, h), jnp.float32),
        mesh=_MESH,
        scratch_types=[
            pltpu.VMEM((CHUNKS_PER_TILE, CHUNK), jnp.int32),
            pltpu.VMEM((CHUNKS_PER_TILE, CHUNK), jnp.int32),
            pltpu.VMEM((_NBUF, CHUNK, h), jnp.float32),
            pltpu.VMEM((CHUNK, h), jnp.float32),
            pltpu.VMEM_SHARED((NP, h), jnp.float32),
        ]
        + [pltpu.SemaphoreType.DMA] * (2 * _NBUF),
        compiler_params=pltpu.CompilerParams(use_tc_tiling_on_sc=False),
    )


N_REAL = 10000


_BLK = 5000
_GRID = N_REAL // _BLK


def _mm_body(x_ref, w_ref, h_ref):
    h_ref[...] = jnp.dot(
        x_ref[...], w_ref[...], preferred_element_type=jnp.float32
    )


def _tc_matmul(x, W1):
    h1w = W1.shape[1]
    return pl.pallas_call(
        _mm_body,
        grid=(_GRID,),
        in_specs=[
            pl.BlockSpec((_BLK, x.shape[1]), lambda i: (i, 0)),
            pl.BlockSpec(W1.shape, lambda i: (0, 0)),
        ],
        out_specs=pl.BlockSpec((_BLK, h1w), lambda i: (i, 0)),
        out_shape=jax.ShapeDtypeStruct((N_REAL, h1w), jnp.float32),
    )(x, W1)


def _k1_body(deg_ref, h_ref, g_ref, dinv_ref, ood_ref):
    deg = deg_ref[0] + deg_ref[1] + 1.0
    dinv = lax.rsqrt(deg)
    ood = 1.0 / deg
    dinv_ref[...] = dinv
    ood_ref[...] = ood
    dcol = dinv.reshape(NP, 1)[:N_REAL]
    g_ref[...] = h_ref[...] * dcol


def _tc_first(deg_parts, h1):
    h1w = h1.shape[1]
    return pl.pallas_call(
        _k1_body,
        grid=(1,),
        in_specs=[
            pl.BlockSpec((NC, NP), lambda i: (0, 0)),
            pl.BlockSpec((N_REAL, h1w), lambda i: (0, 0)),
        ],
        out_specs=[
            pl.BlockSpec((N_REAL, h1w), lambda i: (0, 0)),
            pl.BlockSpec((NP,), lambda i: (0,)),
            pl.BlockSpec((NP,), lambda i: (0,)),
        ],
        out_shape=[
            jax.ShapeDtypeStruct((NP, h1w), jnp.float32),
            jax.ShapeDtypeStruct((NP,), jnp.float32),
            jax.ShapeDtypeStruct((NP,), jnp.float32),
        ],
    )(deg_parts, h1)


def _k2_body(parts_ref, hcur_ref, dinv_ref, ood_ref, b_ref, w_ref,
             hn_ref, gn_ref):
    dcol = dinv_ref[...].reshape(NP, 1)[:N_REAL]
    ocol = ood_ref[...].reshape(NP, 1)[:N_REAL]
    z = (parts_ref[0, :N_REAL] + parts_ref[1, :N_REAL]) * dcol
    z = z + hcur_ref[...] * ocol + b_ref[...]
    z = jnp.maximum(z, 0.0)
    hn = jnp.dot(z, w_ref[...], preferred_element_type=jnp.float32)
    hn_ref[...] = hn
    nw = hn.shape[1]
    gw = gn_ref.shape[1]
    if gw == nw:
        gn_ref[...] = hn * dcol
    else:
        gn_ref[:, :nw] = hn * dcol
        gn_ref[:, nw:] = jnp.zeros((hn.shape[0], gw - nw), jnp.float32)


def _tc_mid(parts, hcur, dinv, ood, b, Wn, gw):
    hw = hcur.shape[1]
    nw = Wn.shape[1]
    return pl.pallas_call(
        _k2_body,
        grid=(1,),
        in_specs=[
            pl.BlockSpec((NC, NP, hw), lambda i: (0, 0, 0)),
            pl.BlockSpec((N_REAL, hw), lambda i: (0, 0)),
            pl.BlockSpec((NP,), lambda i: (0,)),
            pl.BlockSpec((NP,), lambda i: (0,)),
            pl.BlockSpec((1, hw), lambda i: (0, 0)),
            pl.BlockSpec((hw, nw), lambda i: (0, 0)),
        ],
        out_specs=[
            pl.BlockSpec((N_REAL, nw), lambda i: (0, 0)),
            pl.BlockSpec((N_REAL, gw), lambda i: (0, 0)),
        ],
        out_shape=[
            jax.ShapeDtypeStruct((N_REAL, nw), jnp.float32),
            jax.ShapeDtypeStruct((NP, gw), jnp.float32),
        ],
    )(parts, hcur, dinv, ood, b, Wn)


def _k3_body(parts_ref, hcur_ref, dinv_ref, ood_ref, b_ref, out_ref):
    nw = out_ref.shape[1]
    dcol = dinv_ref[...].reshape(NP, 1)[:N_REAL]
    ocol = ood_ref[...].reshape(NP, 1)[:N_REAL]
    logits = (parts_ref[0, :N_REAL, :nw] + parts_ref[1, :N_REAL, :nw]) * dcol
    logits = logits + hcur_ref[...] * ocol + b_ref[...]
    m = jnp.max(logits, axis=-1, keepdims=True)
    e = jnp.exp(logits - m)
    out_ref[...] = e / jnp.sum(e, axis=-1, keepdims=True)


def _tc_last(parts, hcur, dinv, ood, b):
    gw = parts.shape[2]
    nw = hcur.shape[1]
    return pl.pallas_call(
        _k3_body,
        grid=(1,),
        in_specs=[
            pl.BlockSpec((NC, NP, gw), lambda i: (0, 0, 0)),
            pl.BlockSpec((N_REAL, nw), lambda i: (0, 0)),
            pl.BlockSpec((NP,), lambda i: (0,)),
            pl.BlockSpec((NP,), lambda i: (0,)),
            pl.BlockSpec((1, nw), lambda i: (0, 0)),
        ],
        out_specs=pl.BlockSpec((N_REAL, nw), lambda i: (0, 0)),
        out_shape=jax.ShapeDtypeStruct((N_REAL, nw), jnp.float32),
    )(parts, hcur, dinv, ood, b)


def kernel(x, edge_index, W1, b1, W2, b2, W3, b3):
    n, _ = x.shape
    e = edge_index.shape[1]

    nw = NC * NS
    per_w = EP // nw
    pad_w = per_w - e // nw
    pad_idx = jnp.asarray(
        n + (np.arange(nw * pad_w, dtype=np.int32) * 7) % (NP - n),
        dtype=jnp.int32,
    ).reshape(nw, pad_w)
    src3 = jnp.concatenate(
        [edge_index[0].reshape(nw, e // nw), pad_idx], axis=1
    ).reshape(nw, CHUNKS_PER_TILE, CHUNK)
    dst3 = jnp.concatenate(
        [edge_index[1].reshape(nw, e // nw), pad_idx], axis=1
    ).reshape(nw, CHUNKS_PER_TILE, CHUNK)

    deg_parts = _deg_kernel(dst3)

    h1 = _tc_matmul(x, W1)
    g1, dinv, ood = _tc_first(deg_parts, h1)
    parts1 = _edge_kernel(W1.shape[1])(g1, src3, dst3)
    h2, g2 = _tc_mid(parts1, h1, dinv, ood, b1.reshape(1, -1), W2, 64)
    parts2 = _edge_kernel(64)(g2, src3, dst3)
    h3, g3 = _tc_mid(parts2, h2, dinv, ood, b2.reshape(1, -1), W3, 48)
    parts3 = _edge_kernel(48)(g3, src3, dst3)
    return _tc_last(parts3, h3, dinv, ood, b3.reshape(1, -1))


# speedup vs baseline: 24.7364x; 1.0404x over previous
import functools

import numpy as np

import jax
import jax.numpy as jnp
from jax import lax
from jax.experimental import pallas as pl
from jax.experimental.pallas import tpu as pltpu, tpu_sc as plsc

NC = 2
NS = 16
LANES = 16

NP = 10240
ROWS_PER_TILE = NP // NS
CHUNK = 128
CHUNKS_PER_TILE = 40
EP = NC * NS * CHUNKS_PER_TILE * CHUNK

_MESH = plsc.VectorSubcoreMesh(core_axis_name="c", subcore_axis_name="s")


def _deg_body(dst_hbm, out_hbm, dst_v, deg_v, acc_v, tmp_v, shared):
    c = lax.axis_index("c")
    s = lax.axis_index("s")
    w = c * NS + s

    pltpu.sync_copy(dst_hbm.at[w], dst_v)

    zeros16 = jnp.zeros((LANES,), jnp.float32)

    def _zero(j, _):
        deg_v[pl.ds(j * LANES, LANES)] = zeros16
        return 0

    lax.fori_loop(0, NP // LANES, _zero, 0)

    ones16 = jnp.ones((LANES,), jnp.float32)

    def _edges(j, _):
        for k in range(CHUNK // LANES):
            d = dst_v[j, pl.ds(k * LANES, LANES)]
            plsc.addupdate_scatter(deg_v, [d], ones16)
        return 0

    lax.fori_loop(0, CHUNKS_PER_TILE, _edges, 0)

    pltpu.sync_copy(deg_v, shared.at[s])
    plsc.subcore_barrier()

    base = s * ROWS_PER_TILE

    def _zacc(j, _):
        acc_v[pl.ds(j * LANES, LANES)] = zeros16
        return 0

    lax.fori_loop(0, ROWS_PER_TILE // LANES, _zacc, 0)

    for t in range(NS):
        pltpu.sync_copy(shared.at[t, pl.ds(base, ROWS_PER_TILE)], tmp_v)

        def _acc(j, _):
            sl = pl.ds(j * LANES, LANES)
            acc_v[sl] = acc_v[sl] + tmp_v[sl]
            return 0

        lax.fori_loop(0, ROWS_PER_TILE // LANES, _acc, 0)

    pltpu.sync_copy(acc_v, out_hbm.at[c, pl.ds(base, ROWS_PER_TILE)])


_deg_kernel = pl.kernel(
    _deg_body,
    out_type=jax.ShapeDtypeStruct((NC, NP), jnp.float32),
    mesh=_MESH,
    scratch_types=[
        pltpu.VMEM((CHUNKS_PER_TILE, CHUNK), jnp.int32),
        pltpu.VMEM((NP,), jnp.float32),
        pltpu.VMEM((ROWS_PER_TILE,), jnp.float32),
        pltpu.VMEM((ROWS_PER_TILE,), jnp.float32),
        pltpu.VMEM_SHARED((NS, NP), jnp.float32),
    ],
    compiler_params=pltpu.CompilerParams(
        needs_layout_passes=False, use_tc_tiling_on_sc=False
    ),
)


_NBUF = 8


def _edge_body(g_hbm, src_hbm, dst_hbm, out_hbm, src_v, dst_v, rows_v, zbuf_v,
               shared, *sems, h):
    gsems = sems[:_NBUF]
    ssems = sems[_NBUF:]
    c = lax.axis_index("c")
    s = lax.axis_index("s")
    w = c * NS + s

    pltpu.sync_copy(src_hbm.at[w], src_v)
    pltpu.sync_copy(dst_hbm.at[w], dst_v)

    zeros16 = jnp.zeros((LANES,), jnp.float32)

    def _zero(j, _):
        for k in range(h // LANES):
            zbuf_v[j, pl.ds(k * LANES, LANES)] = zeros16
        return 0

    lax.fori_loop(0, CHUNK, _zero, 0)

    base = s * ROWS_PER_TILE
    for i in range(ROWS_PER_TILE // CHUNK):
        pltpu.sync_copy(zbuf_v, shared.at[pl.ds(base + i * CHUNK, CHUNK)])
    plsc.subcore_barrier()

    gd = [None] * _NBUF
    sd = [None] * _NBUF
    for b in range(_NBUF):
        gd[b] = pltpu.async_copy(g_hbm.at[src_v.at[b]], rows_v.at[b], gsems[b])
    for j in range(CHUNKS_PER_TILE):
        b = j % _NBUF
        gd[b].wait()
        sd[b] = pltpu.async_copy(
            rows_v.at[b], shared.at[dst_v.at[j]], ssems[b], add=True
        )
        if j + _NBUF < CHUNKS_PER_TILE:
            sd[b].wait()
            gd[b] = pltpu.async_copy(
                g_hbm.at[src_v.at[j + _NBUF]], rows_v.at[b], gsems[b]
            )
    for j in range(CHUNKS_PER_TILE - _NBUF, CHUNKS_PER_TILE):
        sd[j % _NBUF].wait()

    plsc.subcore_barrier()
    for i in range(ROWS_PER_TILE // CHUNK):
        pltpu.sync_copy(
            shared.at[pl.ds(base + i * CHUNK, CHUNK)],
            out_hbm.at[c, pl.ds(base + i * CHUNK, CHUNK)],
        )


@functools.cache
def _edge_kernel(h):
    return pl.kernel(
        functools.partial(_edge_body, h=h),
        out_type=jax.ShapeDtypeStruct((NC, NP, h), jnp.float32),
        mesh=_MESH,
        scratch_types=[
            pltpu.VMEM((CHUNKS_PER_TILE, CHUNK), jnp.int32),
            pltpu.VMEM((CHUNKS_PER_TILE, CHUNK), jnp.int32),
            pltpu.VMEM((_NBUF, CHUNK, h), jnp.float32),
            pltpu.VMEM((CHUNK, h), jnp.float32),
            pltpu.VMEM_SHARED((NP, h), jnp.float32),
        ]
        + [pltpu.SemaphoreType.DMA] * (2 * _NBUF),
        compiler_params=pltpu.CompilerParams(use_tc_tiling_on_sc=False),
    )


N_REAL = 10000


_BLK = 5000
_GRID = N_REAL // _BLK
_BLK2 = NP // 2


def _mm_body(x_ref, w_ref, h_ref):
    h_ref[...] = jnp.dot(
        x_ref[...], w_ref[...], preferred_element_type=jnp.float32
    )


def _tc_matmul(x, W1):
    h1w = W1.shape[1]
    return pl.pallas_call(
        _mm_body,
        grid=(_GRID,),
        in_specs=[
            pl.BlockSpec((_BLK, x.shape[1]), lambda i: (i, 0)),
            pl.BlockSpec(W1.shape, lambda i: (0, 0)),
        ],
        out_specs=pl.BlockSpec((_BLK, h1w), lambda i: (i, 0)),
        out_shape=jax.ShapeDtypeStruct((NP, h1w), jnp.float32),
    )(x, W1)


def _k1_body(deg_ref, h_ref, g_ref, dinv_ref, ood_ref):
    deg = deg_ref[0] + deg_ref[1] + 1.0
    dinv = lax.rsqrt(deg)
    ood = 1.0 / deg
    dinv_ref[...] = dinv
    ood_ref[...] = ood
    dcol = dinv.reshape(NP, 1)
    g_ref[...] = h_ref[...] * dcol


def _tc_first(deg_parts, h1):
    h1w = h1.shape[1]
    return pl.pallas_call(
        _k1_body,
        grid=(1,),
        in_specs=[
            pl.BlockSpec((NC, NP), lambda i: (0, 0)),
            pl.BlockSpec((NP, h1w), lambda i: (0, 0)),
        ],
        out_specs=[
            pl.BlockSpec((NP, h1w), lambda i: (0, 0)),
            pl.BlockSpec((NP,), lambda i: (0,)),
            pl.BlockSpec((NP,), lambda i: (0,)),
        ],
        out_shape=[
            jax.ShapeDtypeStruct((NP, h1w), jnp.float32),
            jax.ShapeDtypeStruct((NP,), jnp.float32),
            jax.ShapeDtypeStruct((NP,), jnp.float32),
        ],
    )(deg_parts, h1)


def _cols(ref, i):
    return ref[pl.ds(i * _BLK2, _BLK2)].reshape(_BLK2, 1)


def _k2_body(parts_ref, hcur_ref, dinv_ref, ood_ref, b_ref, w_ref,
             hn_ref, gn_ref):
    i = pl.program_id(0)
    dcol = _cols(dinv_ref, i)
    ocol = _cols(ood_ref, i)
    z = (parts_ref[0] + parts_ref[1]) * dcol
    z = z + hcur_ref[...] * ocol + b_ref[...]
    z = jnp.maximum(z, 0.0)
    hn = jnp.dot(z, w_ref[...], preferred_element_type=jnp.float32)
    hn_ref[...] = hn
    nw = hn.shape[1]
    gw = gn_ref.shape[1]
    if gw == nw:
        gn_ref[...] = hn * dcol
    else:
        gn_ref[:, :nw] = hn * dcol
        gn_ref[:, nw:] = jnp.zeros((hn.shape[0], gw - nw), jnp.float32)


def _tc_mid(parts, hcur, dinv, ood, b, Wn, gw):
    hw = hcur.shape[1]
    nw = Wn.shape[1]
    return pl.pallas_call(
        _k2_body,
        grid=(2,),
        in_specs=[
            pl.BlockSpec((NC, _BLK2, hw), lambda i: (0, i, 0)),
            pl.BlockSpec((_BLK2, hw), lambda i: (i, 0)),
            pl.BlockSpec((NP,), lambda i: (0,)),
            pl.BlockSpec((NP,), lambda i: (0,)),
            pl.BlockSpec((1, hw), lambda i: (0, 0)),
            pl.BlockSpec((hw, nw), lambda i: (0, 0)),
        ],
        out_specs=[
            pl.BlockSpec((_BLK2, nw), lambda i: (i, 0)),
            pl.BlockSpec((_BLK2, gw), lambda i: (i, 0)),
        ],
        out_shape=[
            jax.ShapeDtypeStruct((NP, nw), jnp.float32),
            jax.ShapeDtypeStruct((NP, gw), jnp.float32),
        ],
    )(parts, hcur, dinv, ood, b, Wn)


def _k3_body(parts_ref, hcur_ref, dinv_ref, ood_ref, b_ref, out_ref):
    i = pl.program_id(0)
    nw = out_ref.shape[1]
    dcol = _cols(dinv_ref, i)
    ocol = _cols(ood_ref, i)
    logits = (parts_ref[0, :, :nw] + parts_ref[1, :, :nw]) * dcol
    logits = logits + hcur_ref[...] * ocol + b_ref[...]
    m = jnp.max(logits, axis=-1, keepdims=True)
    e = jnp.exp(logits - m)
    out_ref[...] = e / jnp.sum(e, axis=-1, keepdims=True)


def _tc_last(parts, hcur, dinv, ood, b):
    gw = parts.shape[2]
    nw = hcur.shape[1]
    return pl.pallas_call(
        _k3_body,
        grid=(2,),
        in_specs=[
            pl.BlockSpec((NC, _BLK2, gw), lambda i: (0, i, 0)),
            pl.BlockSpec((_BLK2, nw), lambda i: (i, 0)),
            pl.BlockSpec((NP,), lambda i: (0,)),
            pl.BlockSpec((NP,), lambda i: (0,)),
            pl.BlockSpec((1, nw), lambda i: (0, 0)),
        ],
        out_specs=pl.BlockSpec((_BLK2, nw), lambda i: (i, 0)),
        out_shape=jax.ShapeDtypeStruct((NP, nw), jnp.float32),
    )(parts, hcur, dinv, ood, b)


def kernel(x, edge_index, W1, b1, W2, b2, W3, b3):
    n, _ = x.shape
    e = edge_index.shape[1]

    nw = NC * NS
    per_w = EP // nw
    pad_w = per_w - e // nw
    pad_idx = jnp.asarray(
        n + (np.arange(nw * pad_w, dtype=np.int32) * 7) % (NP - n),
        dtype=jnp.int32,
    ).reshape(nw, pad_w)
    src3 = jnp.concatenate(
        [edge_index[0].reshape(nw, e // nw), pad_idx], axis=1
    ).reshape(nw, CHUNKS_PER_TILE, CHUNK)
    dst3 = jnp.concatenate(
        [edge_index[1].reshape(nw, e // nw), pad_idx], axis=1
    ).reshape(nw, CHUNKS_PER_TILE, CHUNK)

    deg_parts = _deg_kernel(dst3)

    h1 = _tc_matmul(x, W1)
    g1, dinv, ood = _tc_first(deg_parts, h1)
    parts1 = _edge_kernel(W1.shape[1])(g1, src3, dst3)
    h2, g2 = _tc_mid(parts1, h1, dinv, ood, b1.reshape(1, -1), W2, 64)
    parts2 = _edge_kernel(64)(g2, src3, dst3)
    h3, g3 = _tc_mid(parts2, h2, dinv, ood, b2.reshape(1, -1), W3, 48)
    parts3 = _edge_kernel(48)(g3, src3, dst3)
    out = _tc_last(parts3, h3, dinv, ood, b3.reshape(1, -1))
    return out[:n]


# speedup vs baseline: 25.4816x; 1.0301x over previous
import functools

import numpy as np

import jax
import jax.numpy as jnp
from jax import lax
from jax.experimental import pallas as pl
from jax.experimental.pallas import tpu as pltpu, tpu_sc as plsc

NC = 2
NS = 16
LANES = 16

NP = 10240
ROWS_PER_TILE = NP // NS
CHUNK = 128
CHUNKS_PER_TILE = 40
EP = NC * NS * CHUNKS_PER_TILE * CHUNK

_MESH = plsc.VectorSubcoreMesh(core_axis_name="c", subcore_axis_name="s")


def _deg_body(ei_hbm, out_hbm, dst_v, deg_v, acc_v, tmp_v, shared):
    c = lax.axis_index("c")
    s = lax.axis_index("s")
    w = c * NS + s

    pltpu.sync_copy(ei_hbm.at[1, w], dst_v)

    zeros16 = jnp.zeros((LANES,), jnp.float32)

    def _zero(j, _):
        deg_v[pl.ds(j * LANES, LANES)] = zeros16
        return 0

    lax.fori_loop(0, NP // LANES, _zero, 0)

    ones16 = jnp.ones((LANES,), jnp.float32)

    def _edges(j, _):
        for k in range(CHUNK // LANES):
            d = dst_v[j, pl.ds(k * LANES, LANES)]
            plsc.addupdate_scatter(deg_v, [d], ones16)
        return 0

    lax.fori_loop(0, CHUNKS_PER_TILE, _edges, 0)

    pltpu.sync_copy(deg_v, shared.at[s])
    plsc.subcore_barrier()

    base = s * ROWS_PER_TILE

    def _zacc(j, _):
        acc_v[pl.ds(j * LANES, LANES)] = zeros16
        return 0

    lax.fori_loop(0, ROWS_PER_TILE // LANES, _zacc, 0)

    for t in range(NS):
        pltpu.sync_copy(shared.at[t, pl.ds(base, ROWS_PER_TILE)], tmp_v)

        def _acc(j, _):
            sl = pl.ds(j * LANES, LANES)
            acc_v[sl] = acc_v[sl] + tmp_v[sl]
            return 0

        lax.fori_loop(0, ROWS_PER_TILE // LANES, _acc, 0)

    pltpu.sync_copy(acc_v, out_hbm.at[c, pl.ds(base, ROWS_PER_TILE)])


_deg_kernel = pl.kernel(
    _deg_body,
    out_type=jax.ShapeDtypeStruct((NC, NP), jnp.float32),
    mesh=_MESH,
    scratch_types=[
        pltpu.VMEM((CHUNKS_PER_TILE, CHUNK), jnp.int32),
        pltpu.VMEM((NP,), jnp.float32),
        pltpu.VMEM((ROWS_PER_TILE,), jnp.float32),
        pltpu.VMEM((ROWS_PER_TILE,), jnp.float32),
        pltpu.VMEM_SHARED((NS, NP), jnp.float32),
    ],
    compiler_params=pltpu.CompilerParams(
        needs_layout_passes=False, use_tc_tiling_on_sc=False
    ),
)


_NBUF = 8


def _edge_body(g_hbm, ei_hbm, out_hbm, src_v, dst_v, rows_v, zbuf_v,
               shared, *sems, h):
    gsems = sems[:_NBUF]
    ssems = sems[_NBUF:]
    c = lax.axis_index("c")
    s = lax.axis_index("s")
    w = c * NS + s

    pltpu.sync_copy(ei_hbm.at[0, w], src_v)
    pltpu.sync_copy(ei_hbm.at[1, w], dst_v)

    zeros16 = jnp.zeros((LANES,), jnp.float32)

    def _zero(j, _):
        for k in range(h // LANES):
            zbuf_v[j, pl.ds(k * LANES, LANES)] = zeros16
        return 0

    lax.fori_loop(0, CHUNK, _zero, 0)

    base = s * ROWS_PER_TILE
    for i in range(ROWS_PER_TILE // CHUNK):
        pltpu.sync_copy(zbuf_v, shared.at[pl.ds(base + i * CHUNK, CHUNK)])
    plsc.subcore_barrier()

    gd = [None] * _NBUF
    sd = [None] * _NBUF
    for b in range(_NBUF):
        gd[b] = pltpu.async_copy(g_hbm.at[src_v.at[b]], rows_v.at[b], gsems[b])
    for j in range(CHUNKS_PER_TILE):
        b = j % _NBUF
        gd[b].wait()
        sd[b] = pltpu.async_copy(
            rows_v.at[b], shared.at[dst_v.at[j]], ssems[b], add=True
        )
        if j + _NBUF < CHUNKS_PER_TILE:
            sd[b].wait()
            gd[b] = pltpu.async_copy(
                g_hbm.at[src_v.at[j + _NBUF]], rows_v.at[b], gsems[b]
            )
    for j in range(CHUNKS_PER_TILE - _NBUF, CHUNKS_PER_TILE):
        sd[j % _NBUF].wait()

    plsc.subcore_barrier()
    for i in range(ROWS_PER_TILE // CHUNK):
        pltpu.sync_copy(
            shared.at[pl.ds(base + i * CHUNK, CHUNK)],
            out_hbm.at[c, pl.ds(base + i * CHUNK, CHUNK)],
        )


@functools.cache
def _edge_kernel(h):
    return pl.kernel(
        functools.partial(_edge_body, h=h),
        out_type=jax.ShapeDtypeStruct((NC, NP, h), jnp.float32),
        mesh=_MESH,
        scratch_types=[
            pltpu.VMEM((CHUNKS_PER_TILE, CHUNK), jnp.int32),
            pltpu.VMEM((CHUNKS_PER_TILE, CHUNK), jnp.int32),
            pltpu.VMEM((_NBUF, CHUNK, h), jnp.float32),
            pltpu.VMEM((CHUNK, h), jnp.float32),
            pltpu.VMEM_SHARED((NP, h), jnp.float32),
        ]
        + [pltpu.SemaphoreType.DMA] * (2 * _NBUF),
        compiler_params=pltpu.CompilerParams(use_tc_tiling_on_sc=False),
    )


N_REAL = 10000


_BLK = 5000
_GRID = N_REAL // _BLK
_BLK2 = NP // 2


def _mm_body(x_ref, w_ref, h_ref):
    h_ref[...] = jnp.dot(
        x_ref[...], w_ref[...], preferred_element_type=jnp.float32
    )


def _tc_matmul(x, W1):
    h1w = W1.shape[1]
    return pl.pallas_call(
        _mm_body,
        grid=(_GRID,),
        in_specs=[
            pl.BlockSpec((_BLK, x.shape[1]), lambda i: (i, 0)),
            pl.BlockSpec(W1.shape, lambda i: (0, 0)),
        ],
        out_specs=pl.BlockSpec((_BLK, h1w), lambda i: (i, 0)),
        out_shape=jax.ShapeDtypeStruct((NP, h1w), jnp.float32),
    )(x, W1)


def _k1_body(deg_ref, h_ref, g_ref, dinv_ref, ood_ref):
    deg = deg_ref[0] + deg_ref[1] + 1.0
    dinv = lax.rsqrt(deg)
    ood = 1.0 / deg
    dinv_ref[...] = dinv
    ood_ref[...] = ood
    dcol = dinv.reshape(NP, 1)
    g_ref[...] = h_ref[...] * dcol


def _tc_first(deg_parts, h1):
    h1w = h1.shape[1]
    return pl.pallas_call(
        _k1_body,
        grid=(1,),
        in_specs=[
            pl.BlockSpec((NC, NP), lambda i: (0, 0)),
            pl.BlockSpec((NP, h1w), lambda i: (0, 0)),
        ],
        out_specs=[
            pl.BlockSpec((NP, h1w), lambda i: (0, 0)),
            pl.BlockSpec((NP,), lambda i: (0,)),
            pl.BlockSpec((NP,), lambda i: (0,)),
        ],
        out_shape=[
            jax.ShapeDtypeStruct((NP, h1w), jnp.float32),
            jax.ShapeDtypeStruct((NP,), jnp.float32),
            jax.ShapeDtypeStruct((NP,), jnp.float32),
        ],
    )(deg_parts, h1)


def _cols(ref, i):
    return ref[pl.ds(i * _BLK2, _BLK2)].reshape(_BLK2, 1)


def _k2_body(parts_ref, hcur_ref, dinv_ref, ood_ref, b_ref, w_ref,
             hn_ref, gn_ref):
    i = pl.program_id(0)
    dcol = _cols(dinv_ref, i)
    ocol = _cols(ood_ref, i)
    z = (parts_ref[0] + parts_ref[1]) * dcol
    z = z + hcur_ref[...] * ocol + b_ref[...]
    z = jnp.maximum(z, 0.0)
    hn = jnp.dot(z, w_ref[...], preferred_element_type=jnp.float32)
    hn_ref[...] = hn
    nw = hn.shape[1]
    gw = gn_ref.shape[1]
    if gw == nw:
        gn_ref[...] = hn * dcol
    else:
        gn_ref[:, :nw] = hn * dcol
        gn_ref[:, nw:] = jnp.zeros((hn.shape[0], gw - nw), jnp.float32)


def _tc_mid(parts, hcur, dinv, ood, b, Wn, gw):
    hw = hcur.shape[1]
    nw = Wn.shape[1]
    return pl.pallas_call(
        _k2_body,
        grid=(2,),
        in_specs=[
            pl.BlockSpec((NC, _BLK2, hw), lambda i: (0, i, 0)),
            pl.BlockSpec((_BLK2, hw), lambda i: (i, 0)),
            pl.BlockSpec((NP,), lambda i: (0,)),
            pl.BlockSpec((NP,), lambda i: (0,)),
            pl.BlockSpec((1, hw), lambda i: (0, 0)),
            pl.BlockSpec((hw, nw), lambda i: (0, 0)),
        ],
        out_specs=[
            pl.BlockSpec((_BLK2, nw), lambda i: (i, 0)),
            pl.BlockSpec((_BLK2, gw), lambda i: (i, 0)),
        ],
        out_shape=[
            jax.ShapeDtypeStruct((NP, nw), jnp.float32),
            jax.ShapeDtypeStruct((NP, gw), jnp.float32),
        ],
    )(parts, hcur, dinv, ood, b, Wn)


def _k3_body(parts_ref, hcur_ref, dinv_ref, ood_ref, b_ref, out_ref):
    i = pl.program_id(0)
    nw = out_ref.shape[1]
    dcol = _cols(dinv_ref, i)
    ocol = _cols(ood_ref, i)
    logits = (parts_ref[0, :, :nw] + parts_ref[1, :, :nw]) * dcol
    logits = logits + hcur_ref[...] * ocol + b_ref[...]
    m = jnp.max(logits, axis=-1, keepdims=True)
    e = jnp.exp(logits - m)
    out_ref[...] = e / jnp.sum(e, axis=-1, keepdims=True)


def _tc_last(parts, hcur, dinv, ood, b):
    gw = parts.shape[2]
    nw = hcur.shape[1]
    return pl.pallas_call(
        _k3_body,
        grid=(2,),
        in_specs=[
            pl.BlockSpec((NC, _BLK2, gw), lambda i: (0, i, 0)),
            pl.BlockSpec((_BLK2, nw), lambda i: (i, 0)),
            pl.BlockSpec((NP,), lambda i: (0,)),
            pl.BlockSpec((NP,), lambda i: (0,)),
            pl.BlockSpec((1, nw), lambda i: (0, 0)),
        ],
        out_specs=pl.BlockSpec((_BLK2, nw), lambda i: (i, 0)),
        out_shape=jax.ShapeDtypeStruct((NP, nw), jnp.float32),
    )(parts, hcur, dinv, ood, b)


def kernel(x, edge_index, W1, b1, W2, b2, W3, b3):
    n, _ = x.shape
    e = edge_index.shape[1]

    nw = NC * NS
    per_w = EP // nw
    pad_w = per_w - e // nw
    pad_idx = jnp.asarray(
        (n + (np.arange(2 * nw * pad_w, dtype=np.int32) * 7) % (NP - n))
        .reshape(2, nw, pad_w),
        dtype=jnp.int32,
    )
    ei3 = jnp.concatenate(
        [edge_index.reshape(2, nw, e // nw), pad_idx], axis=2
    ).reshape(2, nw, CHUNKS_PER_TILE, CHUNK)

    deg_parts = _deg_kernel(ei3)

    h1 = _tc_matmul(x, W1)
    g1, dinv, ood = _tc_first(deg_parts, h1)
    parts1 = _edge_kernel(W1.shape[1])(g1, ei3)
    h2, g2 = _tc_mid(parts1, h1, dinv, ood, b1.reshape(1, -1), W2, 64)
    parts2 = _edge_kernel(64)(g2, ei3)
    h3, g3 = _tc_mid(parts2, h2, dinv, ood, b2.reshape(1, -1), W3, 48)
    parts3 = _edge_kernel(48)(g3, ei3)
    out = _tc_last(parts3, h3, dinv, ood, b3.reshape(1, -1))
    return out[:n]
